# baseline (speedup 1.0000x reference)
import functools

import jax
import jax.numpy as jnp
from jax import lax
from jax.experimental import pallas as pl
from jax.experimental.pallas import tpu as pltpu
from jax.experimental.pallas import tpu_sc as plsc

N = 10000
E = 160000
D_IN = 256
H = 512
L = 3

NC = 2
NS = 16
NW = NC * NS

BN = 1000
CHA = 40
CHB = 80
Q = 128


def _k_in_body(x_ref, w_ref, b_ref, o_ref):
    o_ref[...] = jnp.dot(x_ref[...], w_ref[...],
                         preferred_element_type=jnp.float32) + b_ref[...]


def _tc_in(x, w, b):
    return pl.pallas_call(
        _k_in_body,
        grid=(N // BN,),
        in_specs=[
            pl.BlockSpec((BN, D_IN), lambda i: (i, 0)),
            pl.BlockSpec((D_IN, H), lambda i: (0, 0)),
            pl.BlockSpec((1, H), lambda i: (0, 0)),
        ],
        out_specs=pl.BlockSpec((BN, H), lambda i: (i, 0)),
        out_shape=jax.ShapeDtypeStruct((N, H), jnp.float32),
    )(x, w, b.reshape(1, H))


def _k_proj_body(h_ref, w_ref, b_ref, tm_s_ref, tm_d_ref, ta_s_ref, ta_d_ref):
    y = jnp.dot(h_ref[...], w_ref[...],
                preferred_element_type=jnp.float32) + b_ref[...]
    for q in range(4):
        tm_s_ref[q] = y[:, q * Q:(q + 1) * Q]
        tm_d_ref[q] = y[:, H + q * Q:H + (q + 1) * Q]
    ta_s_ref[...] = y[:, 2 * H:3 * H]
    ta_d_ref[...] = y[:, 3 * H:4 * H]


def _tc_proj(h, wcat, bcat):
    q_spec = pl.BlockSpec((4, BN, Q), lambda i: (0, i, 0))
    f_spec = pl.BlockSpec((BN, H), lambda i: (i, 0))
    return pl.pallas_call(
        _k_proj_body,
        grid=(N // BN,),
        in_specs=[
            pl.BlockSpec((BN, H), lambda i: (i, 0)),
            pl.BlockSpec((H, 4 * H), lambda i: (0, 0)),
            pl.BlockSpec((1, 4 * H), lambda i: (0, 0)),
        ],
        out_specs=[q_spec, q_spec, f_spec, f_spec],
        out_shape=[
            jax.ShapeDtypeStruct((4, N, Q), jnp.float32),
            jax.ShapeDtypeStruct((4, N, Q), jnp.float32),
            jax.ShapeDtypeStruct((N, H), jnp.float32),
            jax.ShapeDtypeStruct((N, H), jnp.float32),
        ],
    )(h, wcat, bcat.reshape(1, 4 * H))


def _k_update_body(agg_ref, h_ref, w1_ref, b1_ref, w2_ref, b2_ref,
                   g_ref, bb_ref, o_ref):
    u = b1_ref[...]
    for q in range(4):
        u = u + jnp.dot(agg_ref[q], w1_ref[q * Q:(q + 1) * Q, :],
                        preferred_element_type=jnp.float32)
    u = u * 0.5 * (1.0 + lax.erf(u * (2.0 ** -0.5)))
    u = jnp.dot(u, w2_ref[...], preferred_element_type=jnp.float32) + b2_ref[...]
    m = jnp.mean(u, axis=-1, keepdims=True)
    va = jnp.mean((u - m) * (u - m), axis=-1, keepdims=True)
    u = (u - m) / jnp.sqrt(va + 1e-5) * g_ref[...] + bb_ref[...]
    o_ref[...] = h_ref[...] + u


def _tc_update(agg, h, w1, b1, w2, b2, g, b):
    return pl.pallas_call(
        _k_update_body,
        grid=(N // BN,),
        in_specs=[
            pl.BlockSpec((4, BN, Q), lambda i: (0, i, 0)),
            pl.BlockSpec((BN, H), lambda i: (i, 0)),
            pl.BlockSpec((H, 2 * H), lambda i: (0, 0)),
            pl.BlockSpec((1, 2 * H), lambda i: (0, 0)),
            pl.BlockSpec((2 * H, H), lambda i: (0, 0)),
            pl.BlockSpec((1, H), lambda i: (0, 0)),
            pl.BlockSpec((1, H), lambda i: (0, 0)),
            pl.BlockSpec((1, H), lambda i: (0, 0)),
        ],
        out_specs=pl.BlockSpec((BN, H), lambda i: (i, 0)),
        out_shape=jax.ShapeDtypeStruct((N, H), jnp.float32),
    )(agg, h, w1, b1.reshape(1, 2 * H), w2, b2.reshape(1, H),
      g.reshape(1, H), b.reshape(1, H))


def _k_ln_body(h_ref, g_ref, b_ref, o_ref):
    v = h_ref[...]
    m = jnp.mean(v, axis=-1, keepdims=True)
    va = jnp.mean((v - m) * (v - m), axis=-1, keepdims=True)
    o_ref[...] = (v - m) / jnp.sqrt(va + 1e-5) * g_ref[...] + b_ref[...]


def _tc_ln(h, g, b):
    return pl.pallas_call(
        _k_ln_body,
        grid=(N // BN,),
        in_specs=[
            pl.BlockSpec((BN, H), lambda i: (i, 0)),
            pl.BlockSpec((1, H), lambda i: (0, 0)),
            pl.BlockSpec((1, H), lambda i: (0, 0)),
        ],
        out_specs=pl.BlockSpec((BN, H), lambda i: (i, 0)),
        out_shape=jax.ShapeDtypeStruct((N, H), jnp.float32),
    )(h, g.reshape(1, H), b.reshape(1, H))


_MESH = plsc.VectorSubcoreMesh(core_axis_name="c", subcore_axis_name="s")

_EPW_A = E // NW
_STEPS_A = _EPW_A // CHA


@functools.partial(
    pl.kernel,
    out_type=jax.ShapeDtypeStruct((E,), jnp.float32),
    mesh=_MESH,
    compiler_params=pltpu.CompilerParams(use_tc_tiling_on_sc=False, needs_layout_passes=False),
    scratch_types=[
        pltpu.VMEM((CHA,), jnp.int32),
        pltpu.VMEM((CHA,), jnp.int32),
        pltpu.VMEM((CHA, H), jnp.float32),
        pltpu.VMEM((CHA, H), jnp.float32),
        pltpu.VMEM((48,), jnp.float32),
        pltpu.VMEM((H,), jnp.float32),
        pltpu.VMEM((16,), jnp.float32),
        pltpu.SemaphoreType.DMA,
    ],
)
def _sc_attn(row_hbm, col_hbm, tas_hbm, tad_hbm, wa2_hbm, ba2_hbm,
             w_hbm, idxr, idxc, srcb, dstb, lb, wa2v, ba2v, sem):
    wid = lax.axis_index("s") * NC + lax.axis_index("c")
    pltpu.sync_copy(wa2_hbm, wa2v)
    pltpu.sync_copy(ba2_hbm, ba2v)

    lane = lax.iota(jnp.int32, 16)

    def step(c, carry):
        base = wid * _EPW_A + c * CHA
        pltpu.sync_copy(row_hbm.at[pl.ds(base, CHA)], idxr)
        pltpu.sync_copy(col_hbm.at[pl.ds(base, CHA)], idxc)
        pltpu.async_copy(tas_hbm.at[idxr], srcb, sem).wait()
        pltpu.async_copy(tad_hbm.at[idxc], dstb, sem).wait()

        def edge(e, carry2):
            acc = jnp.zeros((16,), jnp.float32)
            for k in range(H // 16):
                s = srcb[e, pl.ds(k * 16, 16)] + dstb[e, pl.ds(k * 16, 16)]
                s = jnp.maximum(s, 0.2 * s)
                acc = acc + s * wa2v[pl.ds(k * 16, 16)]
            tot = jnp.sum(acc)
            plsc.store_scatter(lb, [jnp.full((16,), e, jnp.int32)],
                               jnp.full((16,), tot, jnp.float32),
                               mask=lane == 0)
            return carry2

        lax.fori_loop(0, CHA, edge, 0, unroll=False)
        for soff in (0, 16, 32):
            v = lb[pl.ds(soff, 16)] + ba2v[...]
            lb[pl.ds(soff, 16)] = 1.0 / (1.0 + jnp.exp(-v))
        pltpu.sync_copy(lb.at[pl.ds(0, CHA)], w_hbm.at[pl.ds(base, CHA)])
        return carry

    lax.fori_loop(0, _STEPS_A, step, 0, unroll=False)


_EPW_B = E // NS
_STEPS_B = _EPW_B // CHB
_RPS = N // NS


@functools.partial(
    pl.kernel,
    out_type=jax.ShapeDtypeStruct((4, N, Q), jnp.float32),
    mesh=_MESH,
    compiler_params=pltpu.CompilerParams(use_tc_tiling_on_sc=False, needs_layout_passes=False),
    scratch_types=[
        pltpu.VMEM((CHB,), jnp.int32),
        pltpu.VMEM((CHB,), jnp.int32),
        pltpu.VMEM((CHB,), jnp.float32),
        pltpu.VMEM((CHB, Q), jnp.float32),
        pltpu.VMEM((CHB, Q), jnp.float32),
        pltpu.VMEM((CHB, Q), jnp.float32),
        pltpu.VMEM((125, Q), jnp.float32),
        pltpu.VMEM_SHARED((N, Q), jnp.float32),
        pltpu.SemaphoreType.DMA,
    ],
)
def _sc_agg(row_hbm, col_hbm, w_hbm,
            tm_s0, tm_s1, tm_s2, tm_s3, tm_d0, tm_d1, tm_d2, tm_d3,
            agg_hbm, idxr, idxc, wv, srcb, dstb, msgb, zb, acc_sh, sem):
    cc = lax.axis_index("c")
    ss = lax.axis_index("s")

    def zrow(i, carry):
        for k in range(Q // 16):
            zb[i, pl.ds(k * 16, 16)] = jnp.zeros((16,), jnp.float32)
        return carry

    lax.fori_loop(0, 125, zrow, 0, unroll=False)

    def quarter(tsrc, tdst, qidx):
        def zcp(z, carry):
            pltpu.sync_copy(zb, acc_sh.at[pl.ds(ss * _RPS + z * 125, 125)])
            return carry

        lax.fori_loop(0, _RPS // 125, zcp, 0, unroll=False)
        plsc.subcore_barrier()

        def step(c, carry):
            base = ss * _EPW_B + c * CHB
            pltpu.sync_copy(row_hbm.at[pl.ds(base, CHB)], idxr)
            pltpu.sync_copy(col_hbm.at[pl.ds(base, CHB)], idxc)
            pltpu.sync_copy(w_hbm.at[pl.ds(base, CHB)], wv)
            pltpu.async_copy(tsrc.at[idxr], srcb, sem).wait()
            pltpu.async_copy(tdst.at[idxc], dstb, sem).wait()

            def edge(e, carry2):
                wb = plsc.load_gather(wv, [jnp.full((16,), e, jnp.int32)])
                for k in range(Q // 16):
                    msgb[e, pl.ds(k * 16, 16)] = (
                        srcb[e, pl.ds(k * 16, 16)]
                        + dstb[e, pl.ds(k * 16, 16)]) * wb
                return carry2

            lax.fori_loop(0, CHB, edge, 0, unroll=False)
            pltpu.sync_copy(msgb, acc_sh.at[idxc], add=True)
            return carry

        lax.fori_loop(0, _STEPS_B, step, 0, unroll=False)
        plsc.subcore_barrier()
        pltpu.sync_copy(
            acc_sh.at[pl.ds(ss * _RPS, _RPS)],
            agg_hbm.at[qidx, pl.ds(ss * _RPS, _RPS)])
        plsc.subcore_barrier()

    @pl.when(cc == 0)
    def _():
        quarter(tm_s0, tm_d0, 0)
        quarter(tm_s1, tm_d1, 1)

    @pl.when(cc == 1)
    def _():
        quarter(tm_s2, tm_d2, 2)
        quarter(tm_s3, tm_d3, 3)


def kernel(x, edge_index, W_in, b_in, Wm, bm, Wa1, ba1, Wa2, ba2,
           Wu1, bu1, Wu2, bu2, ln_g, ln_b, out_g, out_b):
    row = edge_index[0]
    col = edge_index[1]
    h = _tc_in(x, W_in, b_in)
    for l in range(L):
        wcat = jnp.concatenate(
            [Wm[l][:H], Wm[l][H:], Wa1[l][:H], Wa1[l][H:]], axis=1)
        bcat = jnp.concatenate(
            [jnp.zeros((H,), jnp.float32), bm[l],
             jnp.zeros((H,), jnp.float32), ba1[l]])
        tm_s, tm_d, ta_s, ta_d = _tc_proj(h, wcat, bcat)
        w = _sc_attn(row, col, ta_s, ta_d, Wa2[l],
                     jnp.full((16,), ba2[l], jnp.float32))
        agg = _sc_agg(row, col, w,
                      tm_s[0], tm_s[1], tm_s[2], tm_s[3],
                      tm_d[0], tm_d[1], tm_d[2], tm_d[3])
        h = _tc_update(agg, h, Wu1[l], bu1[l], Wu2[l], bu2[l],
                       ln_g[l], ln_b[l])
    return _tc_ln(h, out_g, out_b)


# speedup vs baseline: 1.2862x; 1.2862x over previous
import functools

import jax
import jax.numpy as jnp
from jax import lax
from jax.experimental import pallas as pl
from jax.experimental.pallas import tpu as pltpu
from jax.experimental.pallas import tpu_sc as plsc

N = 10000
E = 160000
D_IN = 256
H = 512
L = 3

NC = 2
NS = 16
NW = NC * NS

BN = 1000
CHA = 40
CHB = 40
Q = 128


def _k_in_body(x_ref, w_ref, b_ref, o_ref):
    o_ref[...] = jnp.dot(x_ref[...], w_ref[...],
                         preferred_element_type=jnp.float32) + b_ref[...]


def _tc_in(x, w, b):
    return pl.pallas_call(
        _k_in_body,
        grid=(N // BN,),
        in_specs=[
            pl.BlockSpec((BN, D_IN), lambda i: (i, 0)),
            pl.BlockSpec((D_IN, H), lambda i: (0, 0)),
            pl.BlockSpec((1, H), lambda i: (0, 0)),
        ],
        out_specs=pl.BlockSpec((BN, H), lambda i: (i, 0)),
        out_shape=jax.ShapeDtypeStruct((N, H), jnp.float32),
    )(x, w, b.reshape(1, H))


def _k_proj_body(h_ref, w_ref, b_ref, tm_s_ref, tm_d_ref, ta_s_ref, ta_d_ref):
    y = jnp.dot(h_ref[...], w_ref[...],
                preferred_element_type=jnp.float32) + b_ref[...]
    for q in range(4):
        tm_s_ref[q] = y[:, q * Q:(q + 1) * Q]
        tm_d_ref[q] = y[:, H + q * Q:H + (q + 1) * Q]
    ta_s_ref[...] = y[:, 2 * H:3 * H]
    ta_d_ref[...] = y[:, 3 * H:4 * H]


def _tc_proj(h, wcat, bcat):
    q_spec = pl.BlockSpec((4, BN, Q), lambda i: (0, i, 0))
    f_spec = pl.BlockSpec((BN, H), lambda i: (i, 0))
    return pl.pallas_call(
        _k_proj_body,
        grid=(N // BN,),
        in_specs=[
            pl.BlockSpec((BN, H), lambda i: (i, 0)),
            pl.BlockSpec((H, 4 * H), lambda i: (0, 0)),
            pl.BlockSpec((1, 4 * H), lambda i: (0, 0)),
        ],
        out_specs=[q_spec, q_spec, f_spec, f_spec],
        out_shape=[
            jax.ShapeDtypeStruct((4, N, Q), jnp.float32),
            jax.ShapeDtypeStruct((4, N, Q), jnp.float32),
            jax.ShapeDtypeStruct((N, H), jnp.float32),
            jax.ShapeDtypeStruct((N, H), jnp.float32),
        ],
    )(h, wcat, bcat.reshape(1, 4 * H))


def _k_update_body(agg_ref, h_ref, w1_ref, b1_ref, w2_ref, b2_ref,
                   g_ref, bb_ref, o_ref):
    u = b1_ref[...]
    for q in range(4):
        u = u + jnp.dot(agg_ref[q], w1_ref[q * Q:(q + 1) * Q, :],
                        preferred_element_type=jnp.float32)
    u = u * 0.5 * (1.0 + lax.erf(u * (2.0 ** -0.5)))
    u = jnp.dot(u, w2_ref[...], preferred_element_type=jnp.float32) + b2_ref[...]
    m = jnp.mean(u, axis=-1, keepdims=True)
    va = jnp.mean((u - m) * (u - m), axis=-1, keepdims=True)
    u = (u - m) / jnp.sqrt(va + 1e-5) * g_ref[...] + bb_ref[...]
    o_ref[...] = h_ref[...] + u


def _tc_update(agg, h, w1, b1, w2, b2, g, b):
    return pl.pallas_call(
        _k_update_body,
        grid=(N // BN,),
        in_specs=[
            pl.BlockSpec((4, BN, Q), lambda i: (0, i, 0)),
            pl.BlockSpec((BN, H), lambda i: (i, 0)),
            pl.BlockSpec((H, 2 * H), lambda i: (0, 0)),
            pl.BlockSpec((1, 2 * H), lambda i: (0, 0)),
            pl.BlockSpec((2 * H, H), lambda i: (0, 0)),
            pl.BlockSpec((1, H), lambda i: (0, 0)),
            pl.BlockSpec((1, H), lambda i: (0, 0)),
            pl.BlockSpec((1, H), lambda i: (0, 0)),
        ],
        out_specs=pl.BlockSpec((BN, H), lambda i: (i, 0)),
        out_shape=jax.ShapeDtypeStruct((N, H), jnp.float32),
    )(agg, h, w1, b1.reshape(1, 2 * H), w2, b2.reshape(1, H),
      g.reshape(1, H), b.reshape(1, H))


def _k_ln_body(h_ref, g_ref, b_ref, o_ref):
    v = h_ref[...]
    m = jnp.mean(v, axis=-1, keepdims=True)
    va = jnp.mean((v - m) * (v - m), axis=-1, keepdims=True)
    o_ref[...] = (v - m) / jnp.sqrt(va + 1e-5) * g_ref[...] + b_ref[...]


def _tc_ln(h, g, b):
    return pl.pallas_call(
        _k_ln_body,
        grid=(N // BN,),
        in_specs=[
            pl.BlockSpec((BN, H), lambda i: (i, 0)),
            pl.BlockSpec((1, H), lambda i: (0, 0)),
            pl.BlockSpec((1, H), lambda i: (0, 0)),
        ],
        out_specs=pl.BlockSpec((BN, H), lambda i: (i, 0)),
        out_shape=jax.ShapeDtypeStruct((N, H), jnp.float32),
    )(h, g.reshape(1, H), b.reshape(1, H))


_MESH = plsc.VectorSubcoreMesh(core_axis_name="c", subcore_axis_name="s")

_EPW_A = E // NW
_STEPS_A = _EPW_A // CHA


@functools.partial(
    pl.kernel,
    out_type=jax.ShapeDtypeStruct((E // CHA, CHA), jnp.float32),
    mesh=_MESH,
    compiler_params=pltpu.CompilerParams(use_tc_tiling_on_sc=False, needs_layout_passes=False),
    scratch_types=[
        pltpu.VMEM((_STEPS_A, CHA), jnp.int32),
        pltpu.VMEM((_STEPS_A, CHA), jnp.int32),
        pltpu.VMEM((CHA, H), jnp.float32),
        pltpu.VMEM((CHA, H), jnp.float32),
        pltpu.VMEM((48,), jnp.float32),
        pltpu.VMEM((H,), jnp.float32),
        pltpu.VMEM((16,), jnp.float32),
        pltpu.SemaphoreType.DMA,
        pltpu.SemaphoreType.DMA,
    ],
)
def _sc_attn(rowc_hbm, colc_hbm, tas_hbm, tad_hbm, wa2_hbm, ba2_hbm,
             w_hbm, rowb, colb, srcb0, dstb0, lb0,
             wa2v, ba2v, sem0, sem1):
    wid = lax.axis_index("s") * NC + lax.axis_index("c")
    pltpu.sync_copy(wa2_hbm, wa2v)
    pltpu.sync_copy(ba2_hbm, ba2v)
    base = wid * _STEPS_A
    pltpu.sync_copy(rowc_hbm.at[pl.ds(base, _STEPS_A)], rowb)
    pltpu.sync_copy(colc_hbm.at[pl.ds(base, _STEPS_A)], colb)

    lane = lax.iota(jnp.int32, 16)

    def compute(srcb, dstb, lb, c):
        def edge(e, carry2):
            acc = jnp.zeros((16,), jnp.float32)
            for k in range(H // 16):
                s = srcb[e, pl.ds(k * 16, 16)] + dstb[e, pl.ds(k * 16, 16)]
                s = jnp.maximum(s, 0.2 * s)
                acc = acc + s * wa2v[pl.ds(k * 16, 16)]
            tot = jnp.sum(acc)
            plsc.store_scatter(lb, [jnp.full((16,), e, jnp.int32)],
                               jnp.full((16,), tot, jnp.float32),
                               mask=lane == 0)
            return carry2

        lax.fori_loop(0, CHA, edge, 0, unroll=False)
        for soff in (0, 16, 32):
            v = lb[pl.ds(soff, 16)] + ba2v[...]
            lb[pl.ds(soff, 16)] = 1.0 / (1.0 + jnp.exp(-v))
        pltpu.sync_copy(lb.at[pl.ds(0, CHA)], w_hbm.at[base + c])

    def step(c, carry):
        hs = pltpu.async_copy(tas_hbm.at[rowb.at[c]], srcb0, sem0)
        hd = pltpu.async_copy(tad_hbm.at[colb.at[c]], dstb0, sem1)
        hs.wait()
        hd.wait()
        compute(srcb0, dstb0, lb0, c)
        return carry

    lax.fori_loop(0, _STEPS_A, step, 0, unroll=False)


_EPW_B = E // NS
_STEPS_B = _EPW_B // CHB
_RPS = N // NS


@functools.partial(
    pl.kernel,
    out_type=jax.ShapeDtypeStruct((4, N, Q), jnp.float32),
    mesh=_MESH,
    compiler_params=pltpu.CompilerParams(use_tc_tiling_on_sc=False, needs_layout_passes=False),
    scratch_types=[
        pltpu.VMEM((_STEPS_B // 2, CHB), jnp.int32),
        pltpu.VMEM((_STEPS_B // 2, CHB), jnp.float32),
        pltpu.VMEM((CHB,), jnp.int32),
        pltpu.VMEM((CHB,), jnp.int32),
        pltpu.VMEM((CHB,), jnp.int32),
        pltpu.VMEM((CHB,), jnp.int32),
        pltpu.VMEM((CHB, Q), jnp.float32),
        pltpu.VMEM((CHB, Q), jnp.float32),
        pltpu.VMEM((CHB, Q), jnp.float32),
        pltpu.VMEM((CHB, Q), jnp.float32),
        pltpu.VMEM((CHB, Q), jnp.float32),
        pltpu.VMEM((CHB, Q), jnp.float32),
        pltpu.VMEM((48,), jnp.float32),
        pltpu.VMEM_SHARED((N, Q), jnp.float32),
        pltpu.SemaphoreType.DMA,
        pltpu.SemaphoreType.DMA,
        pltpu.SemaphoreType.DMA,
        pltpu.SemaphoreType.DMA,
    ],
)
def _sc_agg(pk_hbm, wc_hbm,
            tm_s0, tm_s1, tm_s2, tm_s3, tm_d0, tm_d1, tm_d2, tm_d3,
            agg_hbm, pkb, wb, idxr0, idxc0, idxr1, idxc1,
            srcb0, dstb0, srcb1, dstb1, msgb0, msgb1, wv48,
            acc_sh, sem0, sem1, sem2, sem3):
    cc = lax.axis_index("c")
    ss = lax.axis_index("s")
    _HS = _STEPS_B // 2

    def unpack(c, idxr, idxc):
        for off in (0, 16, 24):
            pk = pkb[c, pl.ds(off, 16)]
            idxr[pl.ds(off, 16)] = lax.shift_right_logical(pk, 16)
            idxc[pl.ds(off, 16)] = lax.bitwise_and(pk, 0xFFFF)

    def compute(srcb, dstb, msgb, c):
        for off in (0, 16, 24):
            wv48[pl.ds(off, 16)] = wb[c, pl.ds(off, 16)]

        def edge(e, carry2):
            wsp = plsc.load_gather(wv48, [jnp.full((16,), e, jnp.int32)])
            for k in range(Q // 16):
                msgb[e, pl.ds(k * 16, 16)] = (
                    srcb[e, pl.ds(k * 16, 16)]
                    + dstb[e, pl.ds(k * 16, 16)]) * wsp
            return carry2

        lax.fori_loop(0, CHB, edge, 0, unroll=False)

    def quarter(tsrc, tdst, qidx):
        def zrow(i, carry):
            for k in range(Q // 16):
                srcb0[i, pl.ds(k * 16, 16)] = jnp.zeros((16,), jnp.float32)
            return carry

        lax.fori_loop(0, CHB, zrow, 0, unroll=False)

        def zcp(z, carry):
            pltpu.sync_copy(srcb0,
                            acc_sh.at[pl.ds(ss * _RPS + z * CHB, CHB)])
            return carry

        lax.fori_loop(0, _RPS // CHB, zcp, 0, unroll=False)
        pltpu.sync_copy(srcb0.at[pl.ds(0, _RPS - (_RPS // CHB) * CHB)],
                        acc_sh.at[pl.ds(ss * _RPS + (_RPS // CHB) * CHB,
                                        _RPS - (_RPS // CHB) * CHB)])
        plsc.subcore_barrier()

        def do_chunk_pair(c0, c1):
            unpack(c0, idxr0, idxc0)
            unpack(c1, idxr1, idxc1)
            h0s = pltpu.async_copy(tsrc.at[idxr0], srcb0, sem0)
            h0d = pltpu.async_copy(tdst.at[idxc0], dstb0, sem1)
            h1s = pltpu.async_copy(tsrc.at[idxr1], srcb1, sem2)
            h1d = pltpu.async_copy(tdst.at[idxc1], dstb1, sem3)
            h0s.wait()
            h0d.wait()
            compute(srcb0, dstb0, msgb0, c0)
            pltpu.sync_copy(msgb0, acc_sh.at[idxc0], add=True)
            h1s.wait()
            h1d.wait()
            compute(srcb1, dstb1, msgb1, c1)
            pltpu.sync_copy(msgb1, acc_sh.at[idxc1], add=True)

        for half in range(2):
            base = ss * _STEPS_B + half * _HS
            pltpu.sync_copy(pk_hbm.at[pl.ds(base, _HS)], pkb)
            pltpu.sync_copy(wc_hbm.at[pl.ds(base, _HS)], wb)

            def pair(i, carry):
                do_chunk_pair(2 * i, 2 * i + 1)
                return carry

            lax.fori_loop(0, _HS // 2, pair, 0, unroll=False)
            ce = _HS - 1
            unpack(ce, idxr0, idxc0)
            he_s = pltpu.async_copy(tsrc.at[idxr0], srcb0, sem0)
            he_d = pltpu.async_copy(tdst.at[idxc0], dstb0, sem1)
            he_s.wait()
            he_d.wait()
            compute(srcb0, dstb0, msgb0, ce)
            pltpu.sync_copy(msgb0, acc_sh.at[idxc0], add=True)
        plsc.subcore_barrier()
        pltpu.sync_copy(
            acc_sh.at[pl.ds(ss * _RPS, _RPS)],
            agg_hbm.at[qidx, pl.ds(ss * _RPS, _RPS)])
        plsc.subcore_barrier()

    @pl.when(cc == 0)
    def _():
        quarter(tm_s0, tm_d0, 0)
        quarter(tm_s1, tm_d1, 1)

    @pl.when(cc == 1)
    def _():
        quarter(tm_s2, tm_d2, 2)
        quarter(tm_s3, tm_d3, 3)


def kernel(x, edge_index, W_in, b_in, Wm, bm, Wa1, ba1, Wa2, ba2,
           Wu1, bu1, Wu2, bu2, ln_g, ln_b, out_g, out_b):
    row = edge_index[0]
    col = edge_index[1]
    row_a = row.reshape(E // CHA, CHA)
    col_a = col.reshape(E // CHA, CHA)
    pk_b = (jnp.left_shift(row, 16) | col).reshape(E // CHB, CHB)
    h = _tc_in(x, W_in, b_in)
    for l in range(L):
        wcat = jnp.concatenate(
            [Wm[l][:H], Wm[l][H:], Wa1[l][:H], Wa1[l][H:]], axis=1)
        bcat = jnp.concatenate(
            [jnp.zeros((H,), jnp.float32), bm[l],
             jnp.zeros((H,), jnp.float32), ba1[l]])
        tm_s, tm_d, ta_s, ta_d = _tc_proj(h, wcat, bcat)
        w = _sc_attn(row_a, col_a, ta_s, ta_d, Wa2[l],
                     jnp.full((16,), ba2[l], jnp.float32))
        agg = _sc_agg(pk_b, w.reshape(E // CHB, CHB),
                      tm_s[0], tm_s[1], tm_s[2], tm_s[3],
                      tm_d[0], tm_d[1], tm_d[2], tm_d[3])
        h = _tc_update(agg, h, Wu1[l], bu1[l], Wu2[l], bu2[l],
                       ln_g[l], ln_b[l])
    return _tc_ln(h, out_g, out_b)


# speedup vs baseline: 1.3210x; 1.0271x over previous
import functools

import jax
import jax.numpy as jnp
from jax import lax
from jax.experimental import pallas as pl
from jax.experimental.pallas import tpu as pltpu
from jax.experimental.pallas import tpu_sc as plsc

N = 10000
E = 160000
D_IN = 256
H = 512
L = 3

NC = 2
NS = 16
NW = NC * NS

BN = 1000
CHA = 40
CHB = 40
Q = 128


def _k_in_body(x_ref, w_ref, b_ref, o_ref):
    o_ref[...] = jnp.dot(x_ref[...], w_ref[...],
                         preferred_element_type=jnp.float32) + b_ref[...]


def _tc_in(x, w, b):
    return pl.pallas_call(
        _k_in_body,
        grid=(N // BN,),
        in_specs=[
            pl.BlockSpec((BN, D_IN), lambda i: (i, 0)),
            pl.BlockSpec((D_IN, H), lambda i: (0, 0)),
            pl.BlockSpec((1, H), lambda i: (0, 0)),
        ],
        out_specs=pl.BlockSpec((BN, H), lambda i: (i, 0)),
        out_shape=jax.ShapeDtypeStruct((N, H), jnp.float32),
    )(x, w, b.reshape(1, H))


def _k_proj_body(h_ref, w_ref, b_ref, tm_s_ref, tm_d_ref, ta_s_ref, ta_d_ref):
    y = jnp.dot(h_ref[...], w_ref[...],
                preferred_element_type=jnp.float32) + b_ref[...]
    for q in range(4):
        tm_s_ref[q] = y[:, q * Q:(q + 1) * Q]
        tm_d_ref[q] = y[:, H + q * Q:H + (q + 1) * Q]
    ta_s_ref[...] = y[:, 2 * H:3 * H]
    ta_d_ref[...] = y[:, 3 * H:4 * H]


def _tc_proj(h, wcat, bcat):
    q_spec = pl.BlockSpec((4, BN, Q), lambda i: (0, i, 0))
    f_spec = pl.BlockSpec((BN, H), lambda i: (i, 0))
    return pl.pallas_call(
        _k_proj_body,
        grid=(N // BN,),
        in_specs=[
            pl.BlockSpec((BN, H), lambda i: (i, 0)),
            pl.BlockSpec((H, 4 * H), lambda i: (0, 0)),
            pl.BlockSpec((1, 4 * H), lambda i: (0, 0)),
        ],
        out_specs=[q_spec, q_spec, f_spec, f_spec],
        out_shape=[
            jax.ShapeDtypeStruct((4, N, Q), jnp.float32),
            jax.ShapeDtypeStruct((4, N, Q), jnp.float32),
            jax.ShapeDtypeStruct((N, H), jnp.float32),
            jax.ShapeDtypeStruct((N, H), jnp.float32),
        ],
    )(h, wcat, bcat.reshape(1, 4 * H))


def _k_update_body(agg_ref, h_ref, w1_ref, b1_ref, w2_ref, b2_ref,
                   g_ref, bb_ref, o_ref):
    u = b1_ref[...]
    for q in range(4):
        u = u + jnp.dot(agg_ref[q], w1_ref[q * Q:(q + 1) * Q, :],
                        preferred_element_type=jnp.float32)
    u = u * 0.5 * (1.0 + lax.erf(u * (2.0 ** -0.5)))
    u = jnp.dot(u, w2_ref[...], preferred_element_type=jnp.float32) + b2_ref[...]
    m = jnp.mean(u, axis=-1, keepdims=True)
    va = jnp.mean((u - m) * (u - m), axis=-1, keepdims=True)
    u = (u - m) / jnp.sqrt(va + 1e-5) * g_ref[...] + bb_ref[...]
    o_ref[...] = h_ref[...] + u


def _tc_update(agg, h, w1, b1, w2, b2, g, b):
    return pl.pallas_call(
        _k_update_body,
        grid=(N // BN,),
        in_specs=[
            pl.BlockSpec((4, BN, Q), lambda i: (0, i, 0)),
            pl.BlockSpec((BN, H), lambda i: (i, 0)),
            pl.BlockSpec((H, 2 * H), lambda i: (0, 0)),
            pl.BlockSpec((1, 2 * H), lambda i: (0, 0)),
            pl.BlockSpec((2 * H, H), lambda i: (0, 0)),
            pl.BlockSpec((1, H), lambda i: (0, 0)),
            pl.BlockSpec((1, H), lambda i: (0, 0)),
            pl.BlockSpec((1, H), lambda i: (0, 0)),
        ],
        out_specs=pl.BlockSpec((BN, H), lambda i: (i, 0)),
        out_shape=jax.ShapeDtypeStruct((N, H), jnp.float32),
    )(agg, h, w1, b1.reshape(1, 2 * H), w2, b2.reshape(1, H),
      g.reshape(1, H), b.reshape(1, H))


def _k_ln_body(h_ref, g_ref, b_ref, o_ref):
    v = h_ref[...]
    m = jnp.mean(v, axis=-1, keepdims=True)
    va = jnp.mean((v - m) * (v - m), axis=-1, keepdims=True)
    o_ref[...] = (v - m) / jnp.sqrt(va + 1e-5) * g_ref[...] + b_ref[...]


def _tc_ln(h, g, b):
    return pl.pallas_call(
        _k_ln_body,
        grid=(N // BN,),
        in_specs=[
            pl.BlockSpec((BN, H), lambda i: (i, 0)),
            pl.BlockSpec((1, H), lambda i: (0, 0)),
            pl.BlockSpec((1, H), lambda i: (0, 0)),
        ],
        out_specs=pl.BlockSpec((BN, H), lambda i: (i, 0)),
        out_shape=jax.ShapeDtypeStruct((N, H), jnp.float32),
    )(h, g.reshape(1, H), b.reshape(1, H))


_MESH = plsc.VectorSubcoreMesh(core_axis_name="c", subcore_axis_name="s")

_EPW_A = E // NW
_STEPS_A = _EPW_A // CHA


@functools.partial(
    pl.kernel,
    out_type=jax.ShapeDtypeStruct((E // CHA, CHA), jnp.float32),
    mesh=_MESH,
    compiler_params=pltpu.CompilerParams(use_tc_tiling_on_sc=False, needs_layout_passes=False),
    scratch_types=[
        pltpu.VMEM((_STEPS_A, CHA), jnp.int32),
        pltpu.VMEM((_STEPS_A, CHA), jnp.int32),
        pltpu.VMEM((CHA, H), jnp.float32),
        pltpu.VMEM((CHA, H), jnp.float32),
        pltpu.VMEM((48,), jnp.float32),
        pltpu.VMEM((H,), jnp.float32),
        pltpu.VMEM((16,), jnp.float32),
        pltpu.SemaphoreType.DMA,
        pltpu.SemaphoreType.DMA,
    ],
)
def _sc_attn(rowc_hbm, colc_hbm, tas_hbm, tad_hbm, wa2_hbm, ba2_hbm,
             w_hbm, rowb, colb, srcb0, dstb0, lb0,
             wa2v, ba2v, sem0, sem1):
    wid = lax.axis_index("s") * NC + lax.axis_index("c")
    pltpu.sync_copy(wa2_hbm, wa2v)
    pltpu.sync_copy(ba2_hbm, ba2v)
    base = wid * _STEPS_A
    pltpu.sync_copy(rowc_hbm.at[pl.ds(base, _STEPS_A)], rowb)
    pltpu.sync_copy(colc_hbm.at[pl.ds(base, _STEPS_A)], colb)

    lane = lax.iota(jnp.int32, 16)

    def compute(srcb, dstb, lb, c):
        def edge(e, carry2):
            acc = jnp.zeros((16,), jnp.float32)
            for k in range(H // 16):
                s = srcb[e, pl.ds(k * 16, 16)] + dstb[e, pl.ds(k * 16, 16)]
                s = jnp.maximum(s, 0.2 * s)
                acc = acc + s * wa2v[pl.ds(k * 16, 16)]
            tot = jnp.sum(acc)
            plsc.store_scatter(lb, [jnp.full((16,), e, jnp.int32)],
                               jnp.full((16,), tot, jnp.float32),
                               mask=lane == 0)
            return carry2

        lax.fori_loop(0, CHA, edge, 0, unroll=False)
        for soff in (0, 16, 32):
            v = lb[pl.ds(soff, 16)] + ba2v[...]
            lb[pl.ds(soff, 16)] = 1.0 / (1.0 + jnp.exp(-v))
        pltpu.sync_copy(lb.at[pl.ds(0, CHA)], w_hbm.at[base + c])

    def step(c, carry):
        hs = pltpu.async_copy(tas_hbm.at[rowb.at[c]], srcb0, sem0)
        hd = pltpu.async_copy(tad_hbm.at[colb.at[c]], dstb0, sem1)
        hs.wait()
        hd.wait()
        compute(srcb0, dstb0, lb0, c)
        return carry

    lax.fori_loop(0, _STEPS_A, step, 0, unroll=False)


_EPW_B = E // NS
_STEPS_B = _EPW_B // CHB
_RPS = N // NS


@functools.partial(
    pl.kernel,
    out_type=jax.ShapeDtypeStruct((4, N, Q), jnp.float32),
    mesh=_MESH,
    compiler_params=pltpu.CompilerParams(use_tc_tiling_on_sc=False, needs_layout_passes=False),
    scratch_types=[
        pltpu.VMEM((_STEPS_B // 2, CHB), jnp.int32),
        pltpu.VMEM((_STEPS_B // 2, CHB), jnp.float32),
        pltpu.VMEM((CHB,), jnp.int32),
        pltpu.VMEM((CHB,), jnp.int32),
        pltpu.VMEM((CHB,), jnp.int32),
        pltpu.VMEM((CHB,), jnp.int32),
        pltpu.VMEM((CHB, Q), jnp.float32),
        pltpu.VMEM((CHB, Q), jnp.float32),
        pltpu.VMEM((CHB, Q), jnp.float32),
        pltpu.VMEM((CHB, Q), jnp.float32),
        pltpu.VMEM((CHB, Q), jnp.float32),
        pltpu.VMEM((CHB, Q), jnp.float32),
        pltpu.VMEM((48,), jnp.float32),
        pltpu.VMEM_SHARED((N, Q), jnp.float32),
        pltpu.SemaphoreType.DMA,
        pltpu.SemaphoreType.DMA,
        pltpu.SemaphoreType.DMA,
        pltpu.SemaphoreType.DMA,
    ],
)
def _sc_agg(pk_hbm, wc_hbm,
            tm_s0, tm_s1, tm_s2, tm_s3, tm_d0, tm_d1, tm_d2, tm_d3,
            agg_hbm, pkb, wb, idxr0, idxc0, idxr1, idxc1,
            srcb0, dstb0, srcb1, dstb1, msgb0, msgb1, wv48,
            acc_sh, sem0, sem1, sem2, sem3):
    cc = lax.axis_index("c")
    ss = lax.axis_index("s")
    _HS = _STEPS_B // 2

    def unpack(c, idxr, idxc):
        for off in (0, 16, 24):
            pk = pkb[c, pl.ds(off, 16)]
            idxr[pl.ds(off, 16)] = lax.shift_right_logical(pk, 16)
            idxc[pl.ds(off, 16)] = lax.bitwise_and(pk, 0xFFFF)

    def compute(srcb, dstb, msgb, c):
        for off in (0, 16, 24):
            wv48[pl.ds(off, 16)] = wb[c, pl.ds(off, 16)]

        def edge(e, carry2):
            wsp = plsc.load_gather(wv48, [jnp.full((16,), e, jnp.int32)])
            for k in range(Q // 16):
                msgb[e, pl.ds(k * 16, 16)] = (
                    srcb[e, pl.ds(k * 16, 16)]
                    + dstb[e, pl.ds(k * 16, 16)]) * wsp
            return carry2

        lax.fori_loop(0, CHB, edge, 0, unroll=False)

    def quarter(tsrc, tdst, qidx):
        def zrow(i, carry):
            for k in range(Q // 16):
                srcb0[i, pl.ds(k * 16, 16)] = jnp.zeros((16,), jnp.float32)
            return carry

        lax.fori_loop(0, CHB, zrow, 0, unroll=False)

        def zcp(z, carry):
            pltpu.sync_copy(srcb0,
                            acc_sh.at[pl.ds(ss * _RPS + z * CHB, CHB)])
            return carry

        lax.fori_loop(0, _RPS // CHB, zcp, 0, unroll=False)
        pltpu.sync_copy(srcb0.at[pl.ds(0, _RPS - (_RPS // CHB) * CHB)],
                        acc_sh.at[pl.ds(ss * _RPS + (_RPS // CHB) * CHB,
                                        _RPS - (_RPS // CHB) * CHB)])
        plsc.subcore_barrier()

        def do_chunk_pair(c0, c1):
            unpack(c0, idxr0, idxc0)
            unpack(c1, idxr1, idxc1)
            h0s = pltpu.async_copy(tsrc.at[idxr0], srcb0, sem0)
            h0d = pltpu.async_copy(tdst.at[idxc0], dstb0, sem1)
            h1s = pltpu.async_copy(tsrc.at[idxr1], srcb1, sem2)
            h1d = pltpu.async_copy(tdst.at[idxc1], dstb1, sem3)
            h0s.wait()
            h0d.wait()
            compute(srcb0, dstb0, msgb0, c0)
            hsc0 = pltpu.async_copy(msgb0, acc_sh.at[idxc0], add=True,
                                    sem=sem0)
            h1s.wait()
            h1d.wait()
            compute(srcb1, dstb1, msgb1, c1)
            hsc1 = pltpu.async_copy(msgb1, acc_sh.at[idxc1], add=True,
                                    sem=sem1)
            hsc0.wait()
            hsc1.wait()

        for half in range(2):
            base = ss * _STEPS_B + half * _HS
            pltpu.sync_copy(pk_hbm.at[pl.ds(base, _HS)], pkb)
            pltpu.sync_copy(wc_hbm.at[pl.ds(base, _HS)], wb)

            def pair(i, carry):
                do_chunk_pair(2 * i, 2 * i + 1)
                return carry

            lax.fori_loop(0, _HS // 2, pair, 0, unroll=False)
            ce = _HS - 1
            unpack(ce, idxr0, idxc0)
            he_s = pltpu.async_copy(tsrc.at[idxr0], srcb0, sem0)
            he_d = pltpu.async_copy(tdst.at[idxc0], dstb0, sem1)
            he_s.wait()
            he_d.wait()
            compute(srcb0, dstb0, msgb0, ce)
            pltpu.sync_copy(msgb0, acc_sh.at[idxc0], add=True)
        plsc.subcore_barrier()
        pltpu.sync_copy(
            acc_sh.at[pl.ds(ss * _RPS, _RPS)],
            agg_hbm.at[qidx, pl.ds(ss * _RPS, _RPS)])
        plsc.subcore_barrier()

    @pl.when(cc == 0)
    def _():
        quarter(tm_s0, tm_d0, 0)
        quarter(tm_s1, tm_d1, 1)

    @pl.when(cc == 1)
    def _():
        quarter(tm_s2, tm_d2, 2)
        quarter(tm_s3, tm_d3, 3)


def kernel(x, edge_index, W_in, b_in, Wm, bm, Wa1, ba1, Wa2, ba2,
           Wu1, bu1, Wu2, bu2, ln_g, ln_b, out_g, out_b):
    row = edge_index[0]
    col = edge_index[1]
    row_a = row.reshape(E // CHA, CHA)
    col_a = col.reshape(E // CHA, CHA)
    pk_b = (jnp.left_shift(row, 16) | col).reshape(E // CHB, CHB)
    h = _tc_in(x, W_in, b_in)
    for l in range(L):
        wcat = jnp.concatenate(
            [Wm[l][:H], Wm[l][H:], Wa1[l][:H], Wa1[l][H:]], axis=1)
        bcat = jnp.concatenate(
            [jnp.zeros((H,), jnp.float32), bm[l],
             jnp.zeros((H,), jnp.float32), ba1[l]])
        tm_s, tm_d, ta_s, ta_d = _tc_proj(h, wcat, bcat)
        w = _sc_attn(row_a, col_a, ta_s, ta_d, Wa2[l],
                     jnp.full((16,), ba2[l], jnp.float32))
        agg = _sc_agg(pk_b, w.reshape(E // CHB, CHB),
                      tm_s[0], tm_s[1], tm_s[2], tm_s[3],
                      tm_d[0], tm_d[1], tm_d[2], tm_d[3])
        h = _tc_update(agg, h, Wu1[l], bu1[l], Wu2[l], bu2[l],
                       ln_g[l], ln_b[l])
    return _tc_ln(h, out_g, out_b)


# speedup vs baseline: 1.5529x; 1.1756x over previous
import functools

import jax
import jax.numpy as jnp
from jax import lax
from jax.experimental import pallas as pl
from jax.experimental.pallas import tpu as pltpu
from jax.experimental.pallas import tpu_sc as plsc

N = 10000
E = 160000
D_IN = 256
H = 512
L = 3

NC = 2
NS = 16
NW = NC * NS

BN = 1000
CHA = 40
CHB = 40
Q = 128


def _k_in_body(x_ref, w_ref, b_ref, o_ref):
    o_ref[...] = jnp.dot(x_ref[...], w_ref[...],
                         preferred_element_type=jnp.float32) + b_ref[...]


def _tc_in(x, w, b):
    return pl.pallas_call(
        _k_in_body,
        grid=(N // BN,),
        in_specs=[
            pl.BlockSpec((BN, D_IN), lambda i: (i, 0)),
            pl.BlockSpec((D_IN, H), lambda i: (0, 0)),
            pl.BlockSpec((1, H), lambda i: (0, 0)),
        ],
        out_specs=pl.BlockSpec((BN, H), lambda i: (i, 0)),
        out_shape=jax.ShapeDtypeStruct((N, H), jnp.float32),
    )(x, w, b.reshape(1, H))


def _k_proj_body(h_ref, w_ref, b_ref, tm_s_ref, tm_d_ref, ta_s_ref, ta_d_ref):
    y = jnp.dot(h_ref[...], w_ref[...],
                preferred_element_type=jnp.float32) + b_ref[...]
    for q in range(4):
        tm_s_ref[q] = y[:, q * Q:(q + 1) * Q]
        tm_d_ref[q] = y[:, H + q * Q:H + (q + 1) * Q]
    ta_s_ref[...] = y[:, 2 * H:3 * H]
    ta_d_ref[...] = y[:, 3 * H:4 * H]


def _tc_proj(h, wcat, bcat):
    q_spec = pl.BlockSpec((4, BN, Q), lambda i: (0, i, 0))
    f_spec = pl.BlockSpec((BN, H), lambda i: (i, 0))
    return pl.pallas_call(
        _k_proj_body,
        grid=(N // BN,),
        in_specs=[
            pl.BlockSpec((BN, H), lambda i: (i, 0)),
            pl.BlockSpec((H, 4 * H), lambda i: (0, 0)),
            pl.BlockSpec((1, 4 * H), lambda i: (0, 0)),
        ],
        out_specs=[q_spec, q_spec, f_spec, f_spec],
        out_shape=[
            jax.ShapeDtypeStruct((4, N, Q), jnp.float32),
            jax.ShapeDtypeStruct((4, N, Q), jnp.float32),
            jax.ShapeDtypeStruct((N, H), jnp.float32),
            jax.ShapeDtypeStruct((N, H), jnp.float32),
        ],
    )(h, wcat, bcat.reshape(1, 4 * H))


def _k_update_body(agg_ref, tmd_ref, sw_ref, h_ref, w1_ref, b1_ref,
                   w2_ref, b2_ref, g_ref, bb_ref, o_ref):
    u = b1_ref[...]
    sw = sw_ref[...]
    for q in range(4):
        u = u + jnp.dot(agg_ref[q] + tmd_ref[q] * sw,
                        w1_ref[q * Q:(q + 1) * Q, :],
                        preferred_element_type=jnp.float32)
    u = u * 0.5 * (1.0 + lax.erf(u * (2.0 ** -0.5)))
    u = jnp.dot(u, w2_ref[...], preferred_element_type=jnp.float32) + b2_ref[...]
    m = jnp.mean(u, axis=-1, keepdims=True)
    va = jnp.mean((u - m) * (u - m), axis=-1, keepdims=True)
    u = (u - m) / jnp.sqrt(va + 1e-5) * g_ref[...] + bb_ref[...]
    o_ref[...] = h_ref[...] + u


def _tc_update(agg, tmd, sw, h, w1, b1, w2, b2, g, b):
    return pl.pallas_call(
        _k_update_body,
        grid=(N // BN,),
        in_specs=[
            pl.BlockSpec((4, BN, Q), lambda i: (0, i, 0)),
            pl.BlockSpec((4, BN, Q), lambda i: (0, i, 0)),
            pl.BlockSpec((BN, 1), lambda i: (i, 0)),
            pl.BlockSpec((BN, H), lambda i: (i, 0)),
            pl.BlockSpec((H, 2 * H), lambda i: (0, 0)),
            pl.BlockSpec((1, 2 * H), lambda i: (0, 0)),
            pl.BlockSpec((2 * H, H), lambda i: (0, 0)),
            pl.BlockSpec((1, H), lambda i: (0, 0)),
            pl.BlockSpec((1, H), lambda i: (0, 0)),
            pl.BlockSpec((1, H), lambda i: (0, 0)),
        ],
        out_specs=pl.BlockSpec((BN, H), lambda i: (i, 0)),
        out_shape=jax.ShapeDtypeStruct((N, H), jnp.float32),
    )(agg, tmd, sw.reshape(N, 1), h, w1, b1.reshape(1, 2 * H),
      w2, b2.reshape(1, H), g.reshape(1, H), b.reshape(1, H))


def _k_ln_body(h_ref, g_ref, b_ref, o_ref):
    v = h_ref[...]
    m = jnp.mean(v, axis=-1, keepdims=True)
    va = jnp.mean((v - m) * (v - m), axis=-1, keepdims=True)
    o_ref[...] = (v - m) / jnp.sqrt(va + 1e-5) * g_ref[...] + b_ref[...]


def _tc_ln(h, g, b):
    return pl.pallas_call(
        _k_ln_body,
        grid=(N // BN,),
        in_specs=[
            pl.BlockSpec((BN, H), lambda i: (i, 0)),
            pl.BlockSpec((1, H), lambda i: (0, 0)),
            pl.BlockSpec((1, H), lambda i: (0, 0)),
        ],
        out_specs=pl.BlockSpec((BN, H), lambda i: (i, 0)),
        out_shape=jax.ShapeDtypeStruct((N, H), jnp.float32),
    )(h, g.reshape(1, H), b.reshape(1, H))


_MESH = plsc.VectorSubcoreMesh(core_axis_name="c", subcore_axis_name="s")

_EPW_A = E // NW
_STEPS_A = _EPW_A // CHA


@functools.partial(
    pl.kernel,
    out_type=jax.ShapeDtypeStruct((E // CHA, CHA), jnp.float32),
    mesh=_MESH,
    compiler_params=pltpu.CompilerParams(use_tc_tiling_on_sc=False, needs_layout_passes=False),
    scratch_types=[
        pltpu.VMEM((_STEPS_A, CHA), jnp.int32),
        pltpu.VMEM((_STEPS_A, CHA), jnp.int32),
        pltpu.VMEM((CHA, H), jnp.float32),
        pltpu.VMEM((CHA, H), jnp.float32),
        pltpu.VMEM((48,), jnp.float32),
        pltpu.VMEM((H,), jnp.float32),
        pltpu.VMEM((16,), jnp.float32),
        pltpu.SemaphoreType.DMA,
        pltpu.SemaphoreType.DMA,
    ],
)
def _sc_attn(rowc_hbm, colc_hbm, tas_hbm, tad_hbm, wa2_hbm, ba2_hbm,
             w_hbm, rowb, colb, srcb0, dstb0, lb0,
             wa2v, ba2v, sem0, sem1):
    wid = lax.axis_index("s") * NC + lax.axis_index("c")
    pltpu.sync_copy(wa2_hbm, wa2v)
    pltpu.sync_copy(ba2_hbm, ba2v)
    base = wid * _STEPS_A
    pltpu.sync_copy(rowc_hbm.at[pl.ds(base, _STEPS_A)], rowb)
    pltpu.sync_copy(colc_hbm.at[pl.ds(base, _STEPS_A)], colb)

    lane = lax.iota(jnp.int32, 16)

    def compute(srcb, dstb, lb, c):
        def edge(e, carry2):
            acc = jnp.zeros((16,), jnp.float32)
            for k in range(H // 16):
                s = srcb[e, pl.ds(k * 16, 16)] + dstb[e, pl.ds(k * 16, 16)]
                s = jnp.maximum(s, 0.2 * s)
                acc = acc + s * wa2v[pl.ds(k * 16, 16)]
            tot = jnp.sum(acc)
            plsc.store_scatter(lb, [jnp.full((16,), e, jnp.int32)],
                               jnp.full((16,), tot, jnp.float32),
                               mask=lane == 0)
            return carry2

        lax.fori_loop(0, CHA, edge, 0, unroll=False)
        for soff in (0, 16, 32):
            v = lb[pl.ds(soff, 16)] + ba2v[...]
            lb[pl.ds(soff, 16)] = 1.0 / (1.0 + jnp.exp(-v))
        pltpu.sync_copy(lb.at[pl.ds(0, CHA)], w_hbm.at[base + c])

    def step(c, carry):
        hs = pltpu.async_copy(tas_hbm.at[rowb.at[c]], srcb0, sem0)
        hd = pltpu.async_copy(tad_hbm.at[colb.at[c]], dstb0, sem1)
        hs.wait()
        hd.wait()
        compute(srcb0, dstb0, lb0, c)
        return carry

    lax.fori_loop(0, _STEPS_A, step, 0, unroll=False)


_EPW_B = E // NS
_STEPS_B = _EPW_B // CHB
_RPS = N // NS


@functools.partial(
    pl.kernel,
    out_type=[jax.ShapeDtypeStruct((4, N, Q), jnp.float32),
              jax.ShapeDtypeStruct((N,), jnp.float32)],
    mesh=_MESH,
    compiler_params=pltpu.CompilerParams(use_tc_tiling_on_sc=False, needs_layout_passes=False),
    scratch_types=[
        pltpu.VMEM((_STEPS_B // 2, CHB), jnp.int32),
        pltpu.VMEM((_STEPS_B // 2, CHB), jnp.float32),
        pltpu.VMEM((CHB,), jnp.int32),
        pltpu.VMEM((CHB,), jnp.int32),
        pltpu.VMEM((CHB,), jnp.int32),
        pltpu.VMEM((CHB,), jnp.int32),
        pltpu.VMEM((CHB, Q), jnp.float32),
        pltpu.VMEM((CHB, Q), jnp.float32),
        pltpu.VMEM((CHB, Q), jnp.float32),
        pltpu.VMEM((CHB, Q), jnp.float32),
        pltpu.VMEM((48,), jnp.float32),
        pltpu.VMEM((640,), jnp.float32),
        pltpu.VMEM_SHARED((N, Q), jnp.float32),
        pltpu.VMEM_SHARED((N,), jnp.float32),
        pltpu.SemaphoreType.DMA,
        pltpu.SemaphoreType.DMA,
        pltpu.SemaphoreType.DMA,
        pltpu.SemaphoreType.DMA,
    ],
)
def _sc_agg(pk_hbm, wc_hbm,
            tm_s0, tm_s1, tm_s2, tm_s3,
            agg_hbm, sw_hbm, pkb, wb, idxr0, idxc0, idxr1, idxc1,
            srcb0, srcb1, msgb0, msgb1, wv48, zb1,
            acc_sh, acc_w, sem0, sem1, sem2, sem3):
    cc = lax.axis_index("c")
    ss = lax.axis_index("s")
    _HS = _STEPS_B // 2

    def unpack(c, idxr, idxc):
        for off in (0, 16, 24):
            pk = pkb[c, pl.ds(off, 16)]
            idxr[pl.ds(off, 16)] = lax.shift_right_logical(pk, 16)
            idxc[pl.ds(off, 16)] = lax.bitwise_and(pk, 0xFFFF)

    def compute(srcb, msgb, c):
        for off in (0, 16, 24):
            wv48[pl.ds(off, 16)] = wb[c, pl.ds(off, 16)]

        def edge(e, carry2):
            wsp = plsc.load_gather(wv48, [jnp.full((16,), e, jnp.int32)])
            for k in range(Q // 16):
                msgb[e, pl.ds(k * 16, 16)] = srcb[e, pl.ds(k * 16, 16)] * wsp
            return carry2

        lax.fori_loop(0, CHB, edge, 0, unroll=False)

    def quarter(tsrc, qidx, do_sw):
        def zrow(i, carry):
            for k in range(Q // 16):
                srcb0[i, pl.ds(k * 16, 16)] = jnp.zeros((16,), jnp.float32)
            return carry

        lax.fori_loop(0, CHB, zrow, 0, unroll=False)

        def zcp(z, carry):
            pltpu.sync_copy(srcb0,
                            acc_sh.at[pl.ds(ss * _RPS + z * CHB, CHB)])
            return carry

        lax.fori_loop(0, _RPS // CHB, zcp, 0, unroll=False)
        pltpu.sync_copy(srcb0.at[pl.ds(0, _RPS - (_RPS // CHB) * CHB)],
                        acc_sh.at[pl.ds(ss * _RPS + (_RPS // CHB) * CHB,
                                        _RPS - (_RPS // CHB) * CHB)])
        if do_sw:
            for kz in range(40):
                zb1[pl.ds(kz * 16, 16)] = jnp.zeros((16,), jnp.float32)
            pltpu.sync_copy(zb1, acc_w.at[pl.ds(ss * 624, 640)])
        plsc.subcore_barrier()

        def do_chunk_pair(c0, c1):
            unpack(c0, idxr0, idxc0)
            unpack(c1, idxr1, idxc1)
            h0s = pltpu.async_copy(tsrc.at[idxr0], srcb0, sem0)
            h1s = pltpu.async_copy(tsrc.at[idxr1], srcb1, sem2)
            h0s.wait()
            compute(srcb0, msgb0, c0)
            hsc0 = pltpu.async_copy(msgb0, acc_sh.at[idxc0], add=True,
                                    sem=sem1)
            if do_sw:
                pltpu.sync_copy(wv48.at[pl.ds(0, CHB)],
                                acc_w.at[idxc0], add=True)
            h1s.wait()
            compute(srcb1, msgb1, c1)
            hsc1 = pltpu.async_copy(msgb1, acc_sh.at[idxc1], add=True,
                                    sem=sem3)
            if do_sw:
                pltpu.sync_copy(wv48.at[pl.ds(0, CHB)],
                                acc_w.at[idxc1], add=True)
            hsc0.wait()
            hsc1.wait()

        for half in range(2):
            base = ss * _STEPS_B + half * _HS
            pltpu.sync_copy(pk_hbm.at[pl.ds(base, _HS)], pkb)
            pltpu.sync_copy(wc_hbm.at[pl.ds(base, _HS)], wb)

            def pair(i, carry):
                do_chunk_pair(2 * i, 2 * i + 1)
                return carry

            lax.fori_loop(0, _HS // 2, pair, 0, unroll=False)
            ce = _HS - 1
            unpack(ce, idxr0, idxc0)
            he_s = pltpu.async_copy(tsrc.at[idxr0], srcb0, sem0)
            he_s.wait()
            compute(srcb0, msgb0, ce)
            pltpu.sync_copy(msgb0, acc_sh.at[idxc0], add=True)
            if do_sw:
                pltpu.sync_copy(wv48.at[pl.ds(0, CHB)],
                                acc_w.at[idxc0], add=True)
        plsc.subcore_barrier()
        pltpu.sync_copy(
            acc_sh.at[pl.ds(ss * _RPS, _RPS)],
            agg_hbm.at[qidx, pl.ds(ss * _RPS, _RPS)])
        if do_sw:
            pltpu.sync_copy(acc_w.at[pl.ds(ss * 624, 640)],
                            sw_hbm.at[pl.ds(ss * 624, 640)])
        plsc.subcore_barrier()

    @pl.when(cc == 0)
    def _():
        quarter(tm_s0, 0, True)
        quarter(tm_s1, 1, False)

    @pl.when(cc == 1)
    def _():
        quarter(tm_s2, 2, False)
        quarter(tm_s3, 3, False)


def kernel(x, edge_index, W_in, b_in, Wm, bm, Wa1, ba1, Wa2, ba2,
           Wu1, bu1, Wu2, bu2, ln_g, ln_b, out_g, out_b):
    row = edge_index[0]
    col = edge_index[1]
    row_a = row.reshape(E // CHA, CHA)
    col_a = col.reshape(E // CHA, CHA)
    pk_b = (jnp.left_shift(row, 16) | col).reshape(E // CHB, CHB)
    h = _tc_in(x, W_in, b_in)
    for l in range(L):
        wcat = jnp.concatenate(
            [Wm[l][:H], Wm[l][H:], Wa1[l][:H], Wa1[l][H:]], axis=1)
        bcat = jnp.concatenate(
            [jnp.zeros((H,), jnp.float32), bm[l],
             jnp.zeros((H,), jnp.float32), ba1[l]])
        tm_s, tm_d, ta_s, ta_d = _tc_proj(h, wcat, bcat)
        w = _sc_attn(row_a, col_a, ta_s, ta_d, Wa2[l],
                     jnp.full((16,), ba2[l], jnp.float32))
        agg, sw = _sc_agg(pk_b, w.reshape(E // CHB, CHB),
                          tm_s[0], tm_s[1], tm_s[2], tm_s[3])
        h = _tc_update(agg, tm_d, sw, h, Wu1[l], bu1[l], Wu2[l], bu2[l],
                       ln_g[l], ln_b[l])
    return _tc_ln(h, out_g, out_b)


# speedup vs baseline: 1.7061x; 1.0986x over previous
import functools

import jax
import jax.numpy as jnp
from jax import lax
from jax.experimental import pallas as pl
from jax.experimental.pallas import tpu as pltpu
from jax.experimental.pallas import tpu_sc as plsc

N = 10000
E = 160000
D_IN = 256
H = 512
L = 3

NC = 2
NS = 16
NW = NC * NS

BN = 1000
CHA = 40
CHB = 40
Q = 128


def _k_in_body(x_ref, w_ref, b_ref, o_ref):
    o_ref[...] = jnp.dot(x_ref[...], w_ref[...],
                         preferred_element_type=jnp.float32) + b_ref[...]


def _tc_in(x, w, b):
    return pl.pallas_call(
        _k_in_body,
        grid=(N // BN,),
        in_specs=[
            pl.BlockSpec((BN, D_IN), lambda i: (i, 0)),
            pl.BlockSpec((D_IN, H), lambda i: (0, 0)),
            pl.BlockSpec((1, H), lambda i: (0, 0)),
        ],
        out_specs=pl.BlockSpec((BN, H), lambda i: (i, 0)),
        out_shape=jax.ShapeDtypeStruct((N, H), jnp.float32),
    )(x, w, b.reshape(1, H))


def _k_proj_body(h_ref, w_ref, b_ref, tm_s_ref, tm_d_ref, ta_s_ref, ta_d_ref):
    y = jnp.dot(h_ref[...], w_ref[...],
                preferred_element_type=jnp.float32) + b_ref[...]
    for q in range(4):
        tm_s_ref[q] = y[:, q * Q:(q + 1) * Q]
        tm_d_ref[q] = y[:, H + q * Q:H + (q + 1) * Q]
    ta_s_ref[...] = y[:, 2 * H:3 * H].astype(jnp.bfloat16)
    ta_d_ref[...] = y[:, 3 * H:4 * H].astype(jnp.bfloat16)


BNP = 400


def _tc_proj(h, wcat, bcat):
    q_spec = pl.BlockSpec((4, BNP, Q), lambda i: (0, i, 0))
    f_spec = pl.BlockSpec((BNP, H), lambda i: (i, 0))
    return pl.pallas_call(
        _k_proj_body,
        grid=(N // BNP,),
        in_specs=[
            pl.BlockSpec((BNP, H), lambda i: (i, 0)),
            pl.BlockSpec((H, 4 * H), lambda i: (0, 0)),
            pl.BlockSpec((1, 4 * H), lambda i: (0, 0)),
        ],
        out_specs=[q_spec, q_spec, f_spec, f_spec],
        out_shape=[
            jax.ShapeDtypeStruct((4, N, Q), jnp.float32),
            jax.ShapeDtypeStruct((4, N, Q), jnp.float32),
            jax.ShapeDtypeStruct((N, H), jnp.bfloat16),
            jax.ShapeDtypeStruct((N, H), jnp.bfloat16),
        ],
    )(h, wcat, bcat.reshape(1, 4 * H))


def _k_update_body(agg_ref, tmd_ref, sw_ref, h_ref, w1_ref, b1_ref,
                   w2_ref, b2_ref, g_ref, bb_ref, o_ref):
    u = b1_ref[...]
    sw = sw_ref[...]
    for q in range(4):
        u = u + jnp.dot(agg_ref[q] + tmd_ref[q] * sw,
                        w1_ref[q * Q:(q + 1) * Q, :],
                        preferred_element_type=jnp.float32)
    u = u * 0.5 * (1.0 + lax.erf(u * (2.0 ** -0.5)))
    u = jnp.dot(u, w2_ref[...], preferred_element_type=jnp.float32) + b2_ref[...]
    m = jnp.mean(u, axis=-1, keepdims=True)
    va = jnp.mean((u - m) * (u - m), axis=-1, keepdims=True)
    u = (u - m) / jnp.sqrt(va + 1e-5) * g_ref[...] + bb_ref[...]
    o_ref[...] = h_ref[...] + u


def _tc_update(agg, tmd, sw, h, w1, b1, w2, b2, g, b):
    return pl.pallas_call(
        _k_update_body,
        grid=(N // BN,),
        in_specs=[
            pl.BlockSpec((4, BN, Q), lambda i: (0, i, 0)),
            pl.BlockSpec((4, BN, Q), lambda i: (0, i, 0)),
            pl.BlockSpec((BN, 1), lambda i: (i, 0)),
            pl.BlockSpec((BN, H), lambda i: (i, 0)),
            pl.BlockSpec((H, 2 * H), lambda i: (0, 0)),
            pl.BlockSpec((1, 2 * H), lambda i: (0, 0)),
            pl.BlockSpec((2 * H, H), lambda i: (0, 0)),
            pl.BlockSpec((1, H), lambda i: (0, 0)),
            pl.BlockSpec((1, H), lambda i: (0, 0)),
            pl.BlockSpec((1, H), lambda i: (0, 0)),
        ],
        out_specs=pl.BlockSpec((BN, H), lambda i: (i, 0)),
        out_shape=jax.ShapeDtypeStruct((N, H), jnp.float32),
    )(agg, tmd, sw.reshape(N, 1), h, w1, b1.reshape(1, 2 * H),
      w2, b2.reshape(1, H), g.reshape(1, H), b.reshape(1, H))


def _k_ln_body(h_ref, g_ref, b_ref, o_ref):
    v = h_ref[...]
    m = jnp.mean(v, axis=-1, keepdims=True)
    va = jnp.mean((v - m) * (v - m), axis=-1, keepdims=True)
    o_ref[...] = (v - m) / jnp.sqrt(va + 1e-5) * g_ref[...] + b_ref[...]


def _tc_ln(h, g, b):
    return pl.pallas_call(
        _k_ln_body,
        grid=(N // BN,),
        in_specs=[
            pl.BlockSpec((BN, H), lambda i: (i, 0)),
            pl.BlockSpec((1, H), lambda i: (0, 0)),
            pl.BlockSpec((1, H), lambda i: (0, 0)),
        ],
        out_specs=pl.BlockSpec((BN, H), lambda i: (i, 0)),
        out_shape=jax.ShapeDtypeStruct((N, H), jnp.float32),
    )(h, g.reshape(1, H), b.reshape(1, H))


_MESH = plsc.VectorSubcoreMesh(core_axis_name="c", subcore_axis_name="s")

_EPW_A = E // NW
_STEPS_A = _EPW_A // CHA


@functools.partial(
    pl.kernel,
    out_type=jax.ShapeDtypeStruct((E // CHA, CHA), jnp.float32),
    mesh=_MESH,
    compiler_params=pltpu.CompilerParams(use_tc_tiling_on_sc=False, needs_layout_passes=False),
    scratch_types=[
        pltpu.VMEM((_STEPS_A, CHA), jnp.int32),
        pltpu.VMEM((_STEPS_A, CHA), jnp.int32),
        pltpu.VMEM((CHA, H), jnp.bfloat16),
        pltpu.VMEM((CHA, H), jnp.bfloat16),
        pltpu.VMEM((CHA, H), jnp.bfloat16),
        pltpu.VMEM((CHA, H), jnp.bfloat16),
        pltpu.VMEM((48,), jnp.float32),
        pltpu.VMEM((48,), jnp.float32),
        pltpu.VMEM((H,), jnp.float32),
        pltpu.VMEM((16,), jnp.float32),
        pltpu.SemaphoreType.DMA,
        pltpu.SemaphoreType.DMA,
        pltpu.SemaphoreType.DMA,
        pltpu.SemaphoreType.DMA,
    ],
)
def _sc_attn(rowc_hbm, colc_hbm, tas_hbm, tad_hbm, wa2_hbm, ba2_hbm,
             w_hbm, rowb, colb, srcb0, dstb0, srcb1, dstb1, lb0, lb1,
             wa2v, ba2v, sem0, sem1, sem2, sem3):
    wid = lax.axis_index("s") * NC + lax.axis_index("c")
    pltpu.sync_copy(wa2_hbm, wa2v)
    pltpu.sync_copy(ba2_hbm, ba2v)
    base = wid * _STEPS_A
    pltpu.sync_copy(rowc_hbm.at[pl.ds(base, _STEPS_A)], rowb)
    pltpu.sync_copy(colc_hbm.at[pl.ds(base, _STEPS_A)], colb)

    lane = lax.iota(jnp.int32, 16)

    def compute(srcb, dstb, lb, c):
        def edge(e, carry2):
            acc = jnp.zeros((16,), jnp.float32)
            for k in range(H // 32):
                s = srcb[e, pl.ds(k * 32, 32)] + dstb[e, pl.ds(k * 32, 32)]
                se, so = plsc.unpack(s, format=plsc.PackFormat.INTERLEAVED)
                se = jnp.maximum(se, 0.2 * se)
                so = jnp.maximum(so, 0.2 * so)
                acc = acc + se * wa2v[pl.ds(k * 32, 16)]
                acc = acc + so * wa2v[pl.ds(k * 32 + 16, 16)]
            tot = jnp.sum(acc)
            plsc.store_scatter(lb, [jnp.full((16,), e, jnp.int32)],
                               jnp.full((16,), tot, jnp.float32),
                               mask=lane == 0)
            return carry2

        lax.fori_loop(0, CHA, edge, 0, unroll=False)
        for soff in (0, 16, 32):
            v = lb[pl.ds(soff, 16)] + ba2v[...]
            lb[pl.ds(soff, 16)] = 1.0 / (1.0 + jnp.exp(-v))
        pltpu.sync_copy(lb.at[pl.ds(0, CHA)], w_hbm.at[base + c])

    def pair(i, carry):
        c0 = 2 * i
        c1 = 2 * i + 1
        h0s = pltpu.async_copy(tas_hbm.at[rowb.at[c0]], srcb0, sem0)
        h0d = pltpu.async_copy(tad_hbm.at[colb.at[c0]], dstb0, sem1)
        h1s = pltpu.async_copy(tas_hbm.at[rowb.at[c1]], srcb1, sem2)
        h1d = pltpu.async_copy(tad_hbm.at[colb.at[c1]], dstb1, sem3)
        h0s.wait()
        h0d.wait()
        compute(srcb0, dstb0, lb0, c0)
        h1s.wait()
        h1d.wait()
        compute(srcb1, dstb1, lb1, c1)
        return carry

    lax.fori_loop(0, _STEPS_A // 2, pair, 0, unroll=False)
    ce = _STEPS_A - 1
    he_s = pltpu.async_copy(tas_hbm.at[rowb.at[ce]], srcb0, sem0)
    he_d = pltpu.async_copy(tad_hbm.at[colb.at[ce]], dstb0, sem1)
    he_s.wait()
    he_d.wait()
    compute(srcb0, dstb0, lb0, ce)


_EPW_B = E // NS
_STEPS_B = _EPW_B // CHB
_RPS = N // NS


@functools.partial(
    pl.kernel,
    out_type=[jax.ShapeDtypeStruct((4, N, Q), jnp.float32),
              jax.ShapeDtypeStruct((N,), jnp.float32)],
    mesh=_MESH,
    compiler_params=pltpu.CompilerParams(use_tc_tiling_on_sc=False, needs_layout_passes=False),
    scratch_types=[
        pltpu.VMEM((_STEPS_B // 2, CHB), jnp.int32),
        pltpu.VMEM((_STEPS_B // 2, CHB), jnp.float32),
        pltpu.VMEM((CHB,), jnp.int32),
        pltpu.VMEM((CHB,), jnp.int32),
        pltpu.VMEM((CHB,), jnp.int32),
        pltpu.VMEM((CHB,), jnp.int32),
        pltpu.VMEM((CHB, Q), jnp.float32),
        pltpu.VMEM((CHB, Q), jnp.float32),
        pltpu.VMEM((CHB, Q), jnp.float32),
        pltpu.VMEM((CHB, Q), jnp.float32),
        pltpu.VMEM((48,), jnp.float32),
        pltpu.VMEM((640,), jnp.float32),
        pltpu.VMEM_SHARED((N, Q), jnp.float32),
        pltpu.VMEM_SHARED((N,), jnp.float32),
        pltpu.SemaphoreType.DMA,
        pltpu.SemaphoreType.DMA,
        pltpu.SemaphoreType.DMA,
        pltpu.SemaphoreType.DMA,
    ],
)
def _sc_agg(pk_hbm, wc_hbm,
            tm_s0, tm_s1, tm_s2, tm_s3,
            agg_hbm, sw_hbm, pkb, wb, idxr0, idxc0, idxr1, idxc1,
            srcb0, srcb1, msgb0, msgb1, wv48, zb1,
            acc_sh, acc_w, sem0, sem1, sem2, sem3):
    cc = lax.axis_index("c")
    ss = lax.axis_index("s")
    _HS = _STEPS_B // 2

    def unpack(c, idxr, idxc):
        for off in (0, 16, 24):
            pk = pkb[c, pl.ds(off, 16)]
            idxr[pl.ds(off, 16)] = lax.shift_right_logical(pk, 16)
            idxc[pl.ds(off, 16)] = lax.bitwise_and(pk, 0xFFFF)

    def compute(srcb, msgb, c):
        for off in (0, 16, 24):
            wv48[pl.ds(off, 16)] = wb[c, pl.ds(off, 16)]

        def edge(e, carry2):
            wsp = plsc.load_gather(wv48, [jnp.full((16,), e, jnp.int32)])
            for k in range(Q // 16):
                msgb[e, pl.ds(k * 16, 16)] = srcb[e, pl.ds(k * 16, 16)] * wsp
            return carry2

        lax.fori_loop(0, CHB, edge, 0, unroll=False)

    def quarter(tsrc, qidx, do_sw):
        def zrow(i, carry):
            for k in range(Q // 16):
                srcb0[i, pl.ds(k * 16, 16)] = jnp.zeros((16,), jnp.float32)
            return carry

        lax.fori_loop(0, CHB, zrow, 0, unroll=False)

        def zcp(z, carry):
            pltpu.sync_copy(srcb0,
                            acc_sh.at[pl.ds(ss * _RPS + z * CHB, CHB)])
            return carry

        lax.fori_loop(0, _RPS // CHB, zcp, 0, unroll=False)
        pltpu.sync_copy(srcb0.at[pl.ds(0, _RPS - (_RPS // CHB) * CHB)],
                        acc_sh.at[pl.ds(ss * _RPS + (_RPS // CHB) * CHB,
                                        _RPS - (_RPS // CHB) * CHB)])
        if do_sw:
            for kz in range(40):
                zb1[pl.ds(kz * 16, 16)] = jnp.zeros((16,), jnp.float32)
            pltpu.sync_copy(zb1, acc_w.at[pl.ds(ss * 624, 640)])
        plsc.subcore_barrier()

        def do_chunk_pair(c0, c1):
            unpack(c0, idxr0, idxc0)
            unpack(c1, idxr1, idxc1)
            h0s = pltpu.async_copy(tsrc.at[idxr0], srcb0, sem0)
            h1s = pltpu.async_copy(tsrc.at[idxr1], srcb1, sem2)
            h0s.wait()
            compute(srcb0, msgb0, c0)
            hsc0 = pltpu.async_copy(msgb0, acc_sh.at[idxc0], add=True,
                                    sem=sem1)
            if do_sw:
                pltpu.sync_copy(wv48.at[pl.ds(0, CHB)],
                                acc_w.at[idxc0], add=True)
            h1s.wait()
            compute(srcb1, msgb1, c1)
            hsc1 = pltpu.async_copy(msgb1, acc_sh.at[idxc1], add=True,
                                    sem=sem3)
            if do_sw:
                pltpu.sync_copy(wv48.at[pl.ds(0, CHB)],
                                acc_w.at[idxc1], add=True)
            hsc0.wait()
            hsc1.wait()

        for half in range(2):
            base = ss * _STEPS_B + half * _HS
            pltpu.sync_copy(pk_hbm.at[pl.ds(base, _HS)], pkb)
            pltpu.sync_copy(wc_hbm.at[pl.ds(base, _HS)], wb)

            def pair(i, carry):
                do_chunk_pair(2 * i, 2 * i + 1)
                return carry

            lax.fori_loop(0, _HS // 2, pair, 0, unroll=False)
            ce = _HS - 1
            unpack(ce, idxr0, idxc0)
            he_s = pltpu.async_copy(tsrc.at[idxr0], srcb0, sem0)
            he_s.wait()
            compute(srcb0, msgb0, ce)
            pltpu.sync_copy(msgb0, acc_sh.at[idxc0], add=True)
            if do_sw:
                pltpu.sync_copy(wv48.at[pl.ds(0, CHB)],
                                acc_w.at[idxc0], add=True)
        plsc.subcore_barrier()
        pltpu.sync_copy(
            acc_sh.at[pl.ds(ss * _RPS, _RPS)],
            agg_hbm.at[qidx, pl.ds(ss * _RPS, _RPS)])
        if do_sw:
            pltpu.sync_copy(acc_w.at[pl.ds(ss * 624, 640)],
                            sw_hbm.at[pl.ds(ss * 624, 640)])
        plsc.subcore_barrier()

    @pl.when(cc == 0)
    def _():
        quarter(tm_s0, 0, True)
        quarter(tm_s1, 1, False)

    @pl.when(cc == 1)
    def _():
        quarter(tm_s2, 2, False)
        quarter(tm_s3, 3, False)


def kernel(x, edge_index, W_in, b_in, Wm, bm, Wa1, ba1, Wa2, ba2,
           Wu1, bu1, Wu2, bu2, ln_g, ln_b, out_g, out_b):
    row = edge_index[0]
    col = edge_index[1]
    row_a = row.reshape(E // CHA, CHA)
    col_a = col.reshape(E // CHA, CHA)
    pk_b = (jnp.left_shift(row, 16) | col).reshape(E // CHB, CHB)
    h = _tc_in(x, W_in, b_in)
    for l in range(L):
        wcat = jnp.concatenate(
            [Wm[l][:H], Wm[l][H:], Wa1[l][:H], Wa1[l][H:]], axis=1)
        bcat = jnp.concatenate(
            [jnp.zeros((H,), jnp.float32), bm[l],
             jnp.zeros((H,), jnp.float32), ba1[l]])
        tm_s, tm_d, ta_s, ta_d = _tc_proj(h, wcat, bcat)
        wa2_re = Wa2[l].reshape(H // 32, 16, 2).transpose(0, 2, 1).reshape(H)
        w = _sc_attn(row_a, col_a, ta_s, ta_d, wa2_re,
                     jnp.full((16,), ba2[l], jnp.float32))
        agg, sw = _sc_agg(pk_b, w.reshape(E // CHB, CHB),
                          tm_s[0], tm_s[1], tm_s[2], tm_s[3])
        h = _tc_update(agg, tm_d, sw, h, Wu1[l], bu1[l], Wu2[l], bu2[l],
                       ln_g[l], ln_b[l])
    return _tc_ln(h, out_g, out_b)


# speedup vs baseline: 2.1261x; 1.2462x over previous
import functools

import jax
import jax.numpy as jnp
from jax import lax
from jax.experimental import pallas as pl
from jax.experimental.pallas import tpu as pltpu
from jax.experimental.pallas import tpu_sc as plsc

N = 10000
E = 160000
D_IN = 256
H = 512
L = 3

NC = 2
NS = 16
NW = NC * NS

BN = 1000
CHA = 40
CHB = 40
Q = 128


def _k_in_body(x_ref, w_ref, b_ref, o_ref):
    o_ref[...] = jnp.dot(x_ref[...], w_ref[...],
                         preferred_element_type=jnp.float32) + b_ref[...]


def _tc_in(x, w, b):
    return pl.pallas_call(
        _k_in_body,
        grid=(N // BN,),
        in_specs=[
            pl.BlockSpec((BN, D_IN), lambda i: (i, 0)),
            pl.BlockSpec((D_IN, H), lambda i: (0, 0)),
            pl.BlockSpec((1, H), lambda i: (0, 0)),
        ],
        out_specs=pl.BlockSpec((BN, H), lambda i: (i, 0)),
        out_shape=jax.ShapeDtypeStruct((N, H), jnp.float32),
    )(x, w, b.reshape(1, H))


def _k_proj_body(h_ref, w_ref, b_ref, tm_s_ref, tm_d_ref, ta_s_ref, ta_d_ref):
    y = jnp.dot(h_ref[...], w_ref[...],
                preferred_element_type=jnp.float32) + b_ref[...]
    for q in range(4):
        tm_s_ref[q] = y[:, q * Q:(q + 1) * Q].astype(jnp.bfloat16)
        tm_d_ref[q] = y[:, H + q * Q:H + (q + 1) * Q]
    ta_s_ref[...] = y[:, 2 * H:3 * H].astype(jnp.bfloat16)
    ta_d_ref[...] = y[:, 3 * H:4 * H].astype(jnp.bfloat16)


BNP = 400


def _tc_proj(h, wcat, bcat):
    q_spec = pl.BlockSpec((4, BNP, Q), lambda i: (0, i, 0))
    f_spec = pl.BlockSpec((BNP, H), lambda i: (i, 0))
    return pl.pallas_call(
        _k_proj_body,
        grid=(N // BNP,),
        in_specs=[
            pl.BlockSpec((BNP, H), lambda i: (i, 0)),
            pl.BlockSpec((H, 4 * H), lambda i: (0, 0)),
            pl.BlockSpec((1, 4 * H), lambda i: (0, 0)),
        ],
        out_specs=[q_spec, q_spec, f_spec, f_spec],
        out_shape=[
            jax.ShapeDtypeStruct((4, N, Q), jnp.bfloat16),
            jax.ShapeDtypeStruct((4, N, Q), jnp.float32),
            jax.ShapeDtypeStruct((N, H), jnp.bfloat16),
            jax.ShapeDtypeStruct((N, H), jnp.bfloat16),
        ],
    )(h, wcat, bcat.reshape(1, 4 * H))


def _k_update_body(agg_ref, tmd_ref, sw_ref, h_ref, w1_ref, b1_ref,
                   w2_ref, b2_ref, g_ref, bb_ref, o_ref):
    u = b1_ref[...]
    sw = sw_ref[...]
    for q in range(4):
        u = u + jnp.dot(agg_ref[q] + tmd_ref[q] * sw,
                        w1_ref[q * Q:(q + 1) * Q, :],
                        preferred_element_type=jnp.float32)
    u = u * 0.5 * (1.0 + lax.erf(u * (2.0 ** -0.5)))
    u = jnp.dot(u, w2_ref[...], preferred_element_type=jnp.float32) + b2_ref[...]
    m = jnp.mean(u, axis=-1, keepdims=True)
    va = jnp.mean((u - m) * (u - m), axis=-1, keepdims=True)
    u = (u - m) / jnp.sqrt(va + 1e-5) * g_ref[...] + bb_ref[...]
    o_ref[...] = h_ref[...] + u


def _tc_update(agg, tmd, sw, h, w1, b1, w2, b2, g, b):
    return pl.pallas_call(
        _k_update_body,
        grid=(N // BN,),
        in_specs=[
            pl.BlockSpec((4, BN, Q), lambda i: (0, i, 0)),
            pl.BlockSpec((4, BN, Q), lambda i: (0, i, 0)),
            pl.BlockSpec((BN, 1), lambda i: (i, 0)),
            pl.BlockSpec((BN, H), lambda i: (i, 0)),
            pl.BlockSpec((H, 2 * H), lambda i: (0, 0)),
            pl.BlockSpec((1, 2 * H), lambda i: (0, 0)),
            pl.BlockSpec((2 * H, H), lambda i: (0, 0)),
            pl.BlockSpec((1, H), lambda i: (0, 0)),
            pl.BlockSpec((1, H), lambda i: (0, 0)),
            pl.BlockSpec((1, H), lambda i: (0, 0)),
        ],
        out_specs=pl.BlockSpec((BN, H), lambda i: (i, 0)),
        out_shape=jax.ShapeDtypeStruct((N, H), jnp.float32),
    )(agg, tmd, sw.reshape(N, 1), h, w1, b1.reshape(1, 2 * H),
      w2, b2.reshape(1, H), g.reshape(1, H), b.reshape(1, H))


def _k_ln_body(h_ref, g_ref, b_ref, o_ref):
    v = h_ref[...]
    m = jnp.mean(v, axis=-1, keepdims=True)
    va = jnp.mean((v - m) * (v - m), axis=-1, keepdims=True)
    o_ref[...] = (v - m) / jnp.sqrt(va + 1e-5) * g_ref[...] + b_ref[...]


def _tc_ln(h, g, b):
    return pl.pallas_call(
        _k_ln_body,
        grid=(N // BN,),
        in_specs=[
            pl.BlockSpec((BN, H), lambda i: (i, 0)),
            pl.BlockSpec((1, H), lambda i: (0, 0)),
            pl.BlockSpec((1, H), lambda i: (0, 0)),
        ],
        out_specs=pl.BlockSpec((BN, H), lambda i: (i, 0)),
        out_shape=jax.ShapeDtypeStruct((N, H), jnp.float32),
    )(h, g.reshape(1, H), b.reshape(1, H))


_MESH = plsc.VectorSubcoreMesh(core_axis_name="c", subcore_axis_name="s")

_EPW_A = E // NW
_STEPS_A = _EPW_A // CHA


@functools.partial(
    pl.kernel,
    out_type=jax.ShapeDtypeStruct((E // CHA, CHA), jnp.float32),
    mesh=_MESH,
    compiler_params=pltpu.CompilerParams(use_tc_tiling_on_sc=False, needs_layout_passes=False),
    scratch_types=[
        pltpu.VMEM((_STEPS_A, CHA), jnp.int32),
        pltpu.VMEM((_STEPS_A, CHA), jnp.int32),
        pltpu.VMEM((CHA, H), jnp.bfloat16),
        pltpu.VMEM((CHA, H), jnp.bfloat16),
        pltpu.VMEM((CHA, H), jnp.bfloat16),
        pltpu.VMEM((CHA, H), jnp.bfloat16),
        pltpu.VMEM((48,), jnp.float32),
        pltpu.VMEM((48,), jnp.float32),
        pltpu.VMEM((H,), jnp.float32),
        pltpu.VMEM((16,), jnp.float32),
        pltpu.SemaphoreType.DMA,
        pltpu.SemaphoreType.DMA,
        pltpu.SemaphoreType.DMA,
        pltpu.SemaphoreType.DMA,
    ],
)
def _sc_attn(rowc_hbm, colc_hbm, tas_hbm, tad_hbm, wa2_hbm, ba2_hbm,
             w_hbm, rowb, colb, srcb0, dstb0, srcb1, dstb1, lb0, lb1,
             wa2v, ba2v, sem0, sem1, sem2, sem3):
    wid = lax.axis_index("s") * NC + lax.axis_index("c")
    pltpu.sync_copy(wa2_hbm, wa2v)
    pltpu.sync_copy(ba2_hbm, ba2v)
    base = wid * _STEPS_A
    pltpu.sync_copy(rowc_hbm.at[pl.ds(base, _STEPS_A)], rowb)
    pltpu.sync_copy(colc_hbm.at[pl.ds(base, _STEPS_A)], colb)

    lane = lax.iota(jnp.int32, 16)

    def compute(srcb, dstb, lb, c):
        def edge(e, carry2):
            acc = jnp.zeros((16,), jnp.float32)
            for k in range(H // 32):
                s = srcb[e, pl.ds(k * 32, 32)] + dstb[e, pl.ds(k * 32, 32)]
                se, so = plsc.unpack(s, format=plsc.PackFormat.INTERLEAVED)
                se = jnp.maximum(se, 0.2 * se)
                so = jnp.maximum(so, 0.2 * so)
                acc = acc + se * wa2v[pl.ds(k * 32, 16)]
                acc = acc + so * wa2v[pl.ds(k * 32 + 16, 16)]
            tot = jnp.sum(acc)
            plsc.store_scatter(lb, [jnp.full((16,), e, jnp.int32)],
                               jnp.full((16,), tot, jnp.float32),
                               mask=lane == 0)
            return carry2

        lax.fori_loop(0, CHA, edge, 0, unroll=False)
        for soff in (0, 16, 32):
            v = lb[pl.ds(soff, 16)] + ba2v[...]
            lb[pl.ds(soff, 16)] = 1.0 / (1.0 + jnp.exp(-v))
        pltpu.sync_copy(lb.at[pl.ds(0, CHA)], w_hbm.at[base + c])

    def pair(i, carry):
        c0 = 2 * i
        c1 = 2 * i + 1
        h0s = pltpu.async_copy(tas_hbm.at[rowb.at[c0]], srcb0, sem0)
        h0d = pltpu.async_copy(tad_hbm.at[colb.at[c0]], dstb0, sem1)
        h1s = pltpu.async_copy(tas_hbm.at[rowb.at[c1]], srcb1, sem2)
        h1d = pltpu.async_copy(tad_hbm.at[colb.at[c1]], dstb1, sem3)
        h0s.wait()
        h0d.wait()
        compute(srcb0, dstb0, lb0, c0)
        h1s.wait()
        h1d.wait()
        compute(srcb1, dstb1, lb1, c1)
        return carry

    lax.fori_loop(0, _STEPS_A // 2, pair, 0, unroll=False)
    ce = _STEPS_A - 1
    he_s = pltpu.async_copy(tas_hbm.at[rowb.at[ce]], srcb0, sem0)
    he_d = pltpu.async_copy(tad_hbm.at[colb.at[ce]], dstb0, sem1)
    he_s.wait()
    he_d.wait()
    compute(srcb0, dstb0, lb0, ce)


_EPW_B = E // NS
_STEPS_B = _EPW_B // CHB
_RPS = N // NS


@functools.partial(
    pl.kernel,
    out_type=[jax.ShapeDtypeStruct((4, N, Q), jnp.float32),
              jax.ShapeDtypeStruct((N,), jnp.float32)],
    mesh=_MESH,
    compiler_params=pltpu.CompilerParams(use_tc_tiling_on_sc=False, needs_layout_passes=False),
    scratch_types=[
        pltpu.VMEM((_STEPS_B // 2, CHB), jnp.int32),
        pltpu.VMEM((_STEPS_B // 2, CHB), jnp.float32),
        pltpu.VMEM((CHB,), jnp.int32),
        pltpu.VMEM((CHB,), jnp.int32),
        pltpu.VMEM((CHB,), jnp.int32),
        pltpu.VMEM((CHB,), jnp.int32),
        pltpu.VMEM((CHB, Q), jnp.bfloat16),
        pltpu.VMEM((CHB, Q), jnp.bfloat16),
        pltpu.VMEM((CHB, Q), jnp.float32),
        pltpu.VMEM((CHB, Q), jnp.float32),
        pltpu.VMEM((48,), jnp.float32),
        pltpu.VMEM((640,), jnp.float32),
        pltpu.VMEM_SHARED((N, Q), jnp.float32),
        pltpu.VMEM_SHARED((N,), jnp.float32),
        pltpu.SemaphoreType.DMA,
        pltpu.SemaphoreType.DMA,
        pltpu.SemaphoreType.DMA,
        pltpu.SemaphoreType.DMA,
    ],
)
def _sc_agg(pk_hbm, wc_hbm,
            tm_s0, tm_s1, tm_s2, tm_s3,
            agg_hbm, sw_hbm, pkb, wb, idxr0, idxc0, idxr1, idxc1,
            srcb0, srcb1, msgb0, msgb1, wv48, zb1,
            acc_sh, acc_w, sem0, sem1, sem2, sem3):
    cc = lax.axis_index("c")
    ss = lax.axis_index("s")
    _HS = _STEPS_B // 2

    def unpack(c, idxr, idxc):
        for off in (0, 16, 24):
            pk = pkb[c, pl.ds(off, 16)]
            idxr[pl.ds(off, 16)] = lax.shift_right_logical(pk, 16)
            idxc[pl.ds(off, 16)] = lax.bitwise_and(pk, 0xFFFF)

    def compute(srcb, msgb, c):
        for off in (0, 16, 24):
            wv48[pl.ds(off, 16)] = wb[c, pl.ds(off, 16)]

        def edge(e, carry2):
            wsp = plsc.load_gather(wv48, [jnp.full((16,), e, jnp.int32)])
            for k in range(Q // 32):
                ab = srcb[e, pl.ds(k * 32, 32)]
                a, b = plsc.unpack(ab, format=plsc.PackFormat.INTERLEAVED)
                msgb[e, pl.ds(k * 32, 16)] = a * wsp
                msgb[e, pl.ds(k * 32 + 16, 16)] = b * wsp
            return carry2

        lax.fori_loop(0, CHB, edge, 0, unroll=False)

    def quarter(tsrc, qidx, do_sw):
        def zrow(i, carry):
            for k in range(Q // 16):
                msgb0[i, pl.ds(k * 16, 16)] = jnp.zeros((16,), jnp.float32)
            return carry

        lax.fori_loop(0, CHB, zrow, 0, unroll=False)

        def zcp(z, carry):
            pltpu.sync_copy(msgb0,
                            acc_sh.at[pl.ds(ss * _RPS + z * CHB, CHB)])
            return carry

        lax.fori_loop(0, _RPS // CHB, zcp, 0, unroll=False)
        pltpu.sync_copy(msgb0.at[pl.ds(0, _RPS - (_RPS // CHB) * CHB)],
                        acc_sh.at[pl.ds(ss * _RPS + (_RPS // CHB) * CHB,
                                        _RPS - (_RPS // CHB) * CHB)])
        if do_sw:
            for kz in range(40):
                zb1[pl.ds(kz * 16, 16)] = jnp.zeros((16,), jnp.float32)
            pltpu.sync_copy(zb1, acc_w.at[pl.ds(ss * 624, 640)])
        plsc.subcore_barrier()

        def do_chunk_pair(c0, c1):
            unpack(c0, idxr0, idxc0)
            unpack(c1, idxr1, idxc1)
            h0s = pltpu.async_copy(tsrc.at[idxr0], srcb0, sem0)
            h1s = pltpu.async_copy(tsrc.at[idxr1], srcb1, sem2)
            h0s.wait()
            compute(srcb0, msgb0, c0)
            hsc0 = pltpu.async_copy(msgb0, acc_sh.at[idxc0], add=True,
                                    sem=sem1)
            if do_sw:
                pltpu.sync_copy(wv48.at[pl.ds(0, CHB)],
                                acc_w.at[idxc0], add=True)
            h1s.wait()
            compute(srcb1, msgb1, c1)
            hsc1 = pltpu.async_copy(msgb1, acc_sh.at[idxc1], add=True,
                                    sem=sem3)
            if do_sw:
                pltpu.sync_copy(wv48.at[pl.ds(0, CHB)],
                                acc_w.at[idxc1], add=True)
            hsc0.wait()
            hsc1.wait()

        for half in range(2):
            base = ss * _STEPS_B + half * _HS
            pltpu.sync_copy(pk_hbm.at[pl.ds(base, _HS)], pkb)
            pltpu.sync_copy(wc_hbm.at[pl.ds(base, _HS)], wb)

            def pair(i, carry):
                do_chunk_pair(2 * i, 2 * i + 1)
                return carry

            lax.fori_loop(0, _HS // 2, pair, 0, unroll=False)
            ce = _HS - 1
            unpack(ce, idxr0, idxc0)
            he_s = pltpu.async_copy(tsrc.at[idxr0], srcb0, sem0)
            he_s.wait()
            compute(srcb0, msgb0, ce)
            pltpu.sync_copy(msgb0, acc_sh.at[idxc0], add=True)
            if do_sw:
                pltpu.sync_copy(wv48.at[pl.ds(0, CHB)],
                                acc_w.at[idxc0], add=True)
        plsc.subcore_barrier()
        pltpu.sync_copy(
            acc_sh.at[pl.ds(ss * _RPS, _RPS)],
            agg_hbm.at[qidx, pl.ds(ss * _RPS, _RPS)])
        if do_sw:
            pltpu.sync_copy(acc_w.at[pl.ds(ss * 624, 640)],
                            sw_hbm.at[pl.ds(ss * 624, 640)])
        plsc.subcore_barrier()

    @pl.when(cc == 0)
    def _():
        quarter(tm_s0, 0, True)
        quarter(tm_s1, 1, False)

    @pl.when(cc == 1)
    def _():
        quarter(tm_s2, 2, False)
        quarter(tm_s3, 3, False)


def kernel(x, edge_index, W_in, b_in, Wm, bm, Wa1, ba1, Wa2, ba2,
           Wu1, bu1, Wu2, bu2, ln_g, ln_b, out_g, out_b):
    row = edge_index[0]
    col = edge_index[1]
    row_a = row.reshape(E // CHA, CHA)
    col_a = col.reshape(E // CHA, CHA)
    pk_b = (jnp.left_shift(row, 16) | col).reshape(E // CHB, CHB)
    h = _tc_in(x, W_in, b_in)
    pidx = jnp.arange(H).reshape(H // 32, 16, 2).transpose(0, 2, 1).reshape(H)
    for l in range(L):
        wcat = jnp.concatenate(
            [Wm[l][:H], Wm[l][H:][:, pidx], Wa1[l][:H], Wa1[l][H:]], axis=1)
        bcat = jnp.concatenate(
            [jnp.zeros((H,), jnp.float32), bm[l][pidx],
             jnp.zeros((H,), jnp.float32), ba1[l]])
        tm_s, tm_d, ta_s, ta_d = _tc_proj(h, wcat, bcat)
        wa2_re = Wa2[l].reshape(H // 32, 16, 2).transpose(0, 2, 1).reshape(H)
        w = _sc_attn(row_a, col_a, ta_s, ta_d, wa2_re,
                     jnp.full((16,), ba2[l], jnp.float32))
        agg, sw = _sc_agg(pk_b, w.reshape(E // CHB, CHB),
                          tm_s[0], tm_s[1], tm_s[2], tm_s[3])
        h = _tc_update(agg, tm_d, sw, h, Wu1[l][pidx, :], bu1[l],
                       Wu2[l], bu2[l], ln_g[l], ln_b[l])
    return _tc_ln(h, out_g, out_b)


# speedup vs baseline: 2.2602x; 1.0631x over previous
import functools

import jax
import jax.numpy as jnp
from jax import lax
from jax.experimental import pallas as pl
from jax.experimental.pallas import tpu as pltpu
from jax.experimental.pallas import tpu_sc as plsc

N = 10000
E = 160000
D_IN = 256
H = 512
L = 3

NC = 2
NS = 16
NW = NC * NS

BN = 1000
CHA = 40
CHB = 80
Q = 128


def _k_in_body(x_ref, w_ref, b_ref, o_ref):
    o_ref[...] = jnp.dot(x_ref[...], w_ref[...],
                         preferred_element_type=jnp.float32) + b_ref[...]


def _tc_in(x, w, b):
    return pl.pallas_call(
        _k_in_body,
        grid=(N // BN,),
        in_specs=[
            pl.BlockSpec((BN, D_IN), lambda i: (i, 0)),
            pl.BlockSpec((D_IN, H), lambda i: (0, 0)),
            pl.BlockSpec((1, H), lambda i: (0, 0)),
        ],
        out_specs=pl.BlockSpec((BN, H), lambda i: (i, 0)),
        out_shape=jax.ShapeDtypeStruct((N, H), jnp.float32),
    )(x, w, b.reshape(1, H))


def _k_proj_body(h_ref, w_ref, b_ref, tm_s_ref, tm_d_ref, ta_s_ref, ta_d_ref):
    y = jnp.dot(h_ref[...], w_ref[...],
                preferred_element_type=jnp.float32) + b_ref[...]
    for q in range(4):
        tm_s_ref[q] = y[:, q * Q:(q + 1) * Q].astype(jnp.bfloat16)
        tm_d_ref[q] = y[:, H + q * Q:H + (q + 1) * Q]
    ta_s_ref[...] = y[:, 2 * H:3 * H].astype(jnp.bfloat16)
    ta_d_ref[...] = y[:, 3 * H:4 * H].astype(jnp.bfloat16)


BNP = 400


def _tc_proj(h, wcat, bcat):
    q_spec = pl.BlockSpec((4, BNP, Q), lambda i: (0, i, 0))
    f_spec = pl.BlockSpec((BNP, H), lambda i: (i, 0))
    return pl.pallas_call(
        _k_proj_body,
        grid=(N // BNP,),
        in_specs=[
            pl.BlockSpec((BNP, H), lambda i: (i, 0)),
            pl.BlockSpec((H, 4 * H), lambda i: (0, 0)),
            pl.BlockSpec((1, 4 * H), lambda i: (0, 0)),
        ],
        out_specs=[q_spec, q_spec, f_spec, f_spec],
        out_shape=[
            jax.ShapeDtypeStruct((4, N, Q), jnp.bfloat16),
            jax.ShapeDtypeStruct((4, N, Q), jnp.float32),
            jax.ShapeDtypeStruct((N, H), jnp.bfloat16),
            jax.ShapeDtypeStruct((N, H), jnp.bfloat16),
        ],
    )(h, wcat, bcat.reshape(1, 4 * H))


def _k_update_body(agg_ref, tmd_ref, sw_ref, h_ref, w1_ref, b1_ref,
                   w2_ref, b2_ref, g_ref, bb_ref, o_ref):
    u = b1_ref[...]
    sw = sw_ref[...]
    for q in range(4):
        u = u + jnp.dot(agg_ref[q] + tmd_ref[q] * sw,
                        w1_ref[q * Q:(q + 1) * Q, :],
                        preferred_element_type=jnp.float32)
    u = u * 0.5 * (1.0 + lax.erf(u * (2.0 ** -0.5)))
    u = jnp.dot(u, w2_ref[...], preferred_element_type=jnp.float32) + b2_ref[...]
    m = jnp.mean(u, axis=-1, keepdims=True)
    va = jnp.mean((u - m) * (u - m), axis=-1, keepdims=True)
    u = (u - m) / jnp.sqrt(va + 1e-5) * g_ref[...] + bb_ref[...]
    o_ref[...] = h_ref[...] + u


def _tc_update(agg, tmd, sw, h, w1, b1, w2, b2, g, b):
    return pl.pallas_call(
        _k_update_body,
        grid=(N // BN,),
        in_specs=[
            pl.BlockSpec((4, BN, Q), lambda i: (0, i, 0)),
            pl.BlockSpec((4, BN, Q), lambda i: (0, i, 0)),
            pl.BlockSpec((BN, 1), lambda i: (i, 0)),
            pl.BlockSpec((BN, H), lambda i: (i, 0)),
            pl.BlockSpec((H, 2 * H), lambda i: (0, 0)),
            pl.BlockSpec((1, 2 * H), lambda i: (0, 0)),
            pl.BlockSpec((2 * H, H), lambda i: (0, 0)),
            pl.BlockSpec((1, H), lambda i: (0, 0)),
            pl.BlockSpec((1, H), lambda i: (0, 0)),
            pl.BlockSpec((1, H), lambda i: (0, 0)),
        ],
        out_specs=pl.BlockSpec((BN, H), lambda i: (i, 0)),
        out_shape=jax.ShapeDtypeStruct((N, H), jnp.float32),
    )(agg, tmd, sw.reshape(N, 1), h, w1, b1.reshape(1, 2 * H),
      w2, b2.reshape(1, H), g.reshape(1, H), b.reshape(1, H))


def _k_ln_body(h_ref, g_ref, b_ref, o_ref):
    v = h_ref[...]
    m = jnp.mean(v, axis=-1, keepdims=True)
    va = jnp.mean((v - m) * (v - m), axis=-1, keepdims=True)
    o_ref[...] = (v - m) / jnp.sqrt(va + 1e-5) * g_ref[...] + b_ref[...]


def _tc_ln(h, g, b):
    return pl.pallas_call(
        _k_ln_body,
        grid=(N // BN,),
        in_specs=[
            pl.BlockSpec((BN, H), lambda i: (i, 0)),
            pl.BlockSpec((1, H), lambda i: (0, 0)),
            pl.BlockSpec((1, H), lambda i: (0, 0)),
        ],
        out_specs=pl.BlockSpec((BN, H), lambda i: (i, 0)),
        out_shape=jax.ShapeDtypeStruct((N, H), jnp.float32),
    )(h, g.reshape(1, H), b.reshape(1, H))


_MESH = plsc.VectorSubcoreMesh(core_axis_name="c", subcore_axis_name="s")

_EPW_A = E // NW
_STEPS_A = _EPW_A // CHA


@functools.partial(
    pl.kernel,
    out_type=jax.ShapeDtypeStruct((E // CHA, CHA), jnp.float32),
    mesh=_MESH,
    compiler_params=pltpu.CompilerParams(use_tc_tiling_on_sc=False, needs_layout_passes=False),
    scratch_types=[
        pltpu.VMEM((_STEPS_A, CHA), jnp.int32),
        pltpu.VMEM((_STEPS_A, CHA), jnp.int32),
        pltpu.VMEM((CHA, H), jnp.bfloat16),
        pltpu.VMEM((CHA, H), jnp.bfloat16),
        pltpu.VMEM((CHA, H), jnp.bfloat16),
        pltpu.VMEM((CHA, H), jnp.bfloat16),
        pltpu.VMEM((48,), jnp.float32),
        pltpu.VMEM((48,), jnp.float32),
        pltpu.VMEM((H,), jnp.float32),
        pltpu.VMEM((16,), jnp.float32),
        pltpu.SemaphoreType.DMA,
        pltpu.SemaphoreType.DMA,
        pltpu.SemaphoreType.DMA,
        pltpu.SemaphoreType.DMA,
    ],
)
def _sc_attn(rowc_hbm, colc_hbm, tas_hbm, tad_hbm, wa2_hbm, ba2_hbm,
             w_hbm, rowb, colb, srcb0, dstb0, srcb1, dstb1, lb0, lb1,
             wa2v, ba2v, sem0, sem1, sem2, sem3):
    wid = lax.axis_index("s") * NC + lax.axis_index("c")
    pltpu.sync_copy(wa2_hbm, wa2v)
    pltpu.sync_copy(ba2_hbm, ba2v)
    base = wid * _STEPS_A
    pltpu.sync_copy(rowc_hbm.at[pl.ds(base, _STEPS_A)], rowb)
    pltpu.sync_copy(colc_hbm.at[pl.ds(base, _STEPS_A)], colb)

    lane = lax.iota(jnp.int32, 16)

    def compute(srcb, dstb, lb, c):
        def edge(e, carry2):
            acc = jnp.zeros((16,), jnp.float32)
            for k in range(H // 32):
                s = srcb[e, pl.ds(k * 32, 32)] + dstb[e, pl.ds(k * 32, 32)]
                se, so = plsc.unpack(s, format=plsc.PackFormat.INTERLEAVED)
                se = jnp.maximum(se, 0.2 * se)
                so = jnp.maximum(so, 0.2 * so)
                acc = acc + se * wa2v[pl.ds(k * 32, 16)]
                acc = acc + so * wa2v[pl.ds(k * 32 + 16, 16)]
            tot = jnp.sum(acc)
            plsc.store_scatter(lb, [jnp.full((16,), e, jnp.int32)],
                               jnp.full((16,), tot, jnp.float32),
                               mask=lane == 0)
            return carry2

        lax.fori_loop(0, CHA, edge, 0, unroll=False)
        for soff in (0, 16, 32):
            v = lb[pl.ds(soff, 16)] + ba2v[...]
            lb[pl.ds(soff, 16)] = 1.0 / (1.0 + jnp.exp(-v))
        pltpu.sync_copy(lb.at[pl.ds(0, CHA)], w_hbm.at[base + c])

    def pair(i, carry):
        c0 = 2 * i
        c1 = 2 * i + 1
        h0s = pltpu.async_copy(tas_hbm.at[rowb.at[c0]], srcb0, sem0)
        h0d = pltpu.async_copy(tad_hbm.at[colb.at[c0]], dstb0, sem1)
        h1s = pltpu.async_copy(tas_hbm.at[rowb.at[c1]], srcb1, sem2)
        h1d = pltpu.async_copy(tad_hbm.at[colb.at[c1]], dstb1, sem3)
        h0s.wait()
        h0d.wait()
        compute(srcb0, dstb0, lb0, c0)
        h1s.wait()
        h1d.wait()
        compute(srcb1, dstb1, lb1, c1)
        return carry

    lax.fori_loop(0, _STEPS_A // 2, pair, 0, unroll=False)
    ce = _STEPS_A - 1
    he_s = pltpu.async_copy(tas_hbm.at[rowb.at[ce]], srcb0, sem0)
    he_d = pltpu.async_copy(tad_hbm.at[colb.at[ce]], dstb0, sem1)
    he_s.wait()
    he_d.wait()
    compute(srcb0, dstb0, lb0, ce)


_EPW_B = E // NS
_STEPS_B = _EPW_B // CHB
_RPS = N // NS


@functools.partial(
    pl.kernel,
    out_type=[jax.ShapeDtypeStruct((4, N, Q), jnp.float32),
              jax.ShapeDtypeStruct((N,), jnp.float32)],
    mesh=_MESH,
    compiler_params=pltpu.CompilerParams(use_tc_tiling_on_sc=False, needs_layout_passes=False),
    scratch_types=[
        pltpu.VMEM((63, CHB), jnp.int32),
        pltpu.VMEM((63, CHB), jnp.float32),
        pltpu.VMEM((CHB,), jnp.int32),
        pltpu.VMEM((CHB,), jnp.int32),
        pltpu.VMEM((CHB,), jnp.int32),
        pltpu.VMEM((CHB,), jnp.int32),
        pltpu.VMEM((CHB, Q), jnp.bfloat16),
        pltpu.VMEM((CHB, Q), jnp.bfloat16),
        pltpu.VMEM((CHB, Q), jnp.float32),
        pltpu.VMEM((CHB, Q), jnp.float32),
        pltpu.VMEM((CHB,), jnp.float32),
        pltpu.VMEM((640,), jnp.float32),
        pltpu.VMEM_SHARED((N, Q), jnp.float32),
        pltpu.VMEM_SHARED((N,), jnp.float32),
        pltpu.SemaphoreType.DMA,
        pltpu.SemaphoreType.DMA,
        pltpu.SemaphoreType.DMA,
        pltpu.SemaphoreType.DMA,
    ],
)
def _sc_agg(pk_hbm, wc_hbm,
            tm_s0, tm_s1, tm_s2, tm_s3,
            agg_hbm, sw_hbm, pkb, wb, idxr0, idxc0, idxr1, idxc1,
            srcb0, srcb1, msgb0, msgb1, wv48, zb1,
            acc_sh, acc_w, sem0, sem1, sem2, sem3):
    cc = lax.axis_index("c")
    ss = lax.axis_index("s")


    def unpack(c, idxr, idxc):
        for off in range(0, CHB, 16):
            pk = pkb[c, pl.ds(off, 16)]
            idxr[pl.ds(off, 16)] = lax.shift_right_logical(pk, 16)
            idxc[pl.ds(off, 16)] = lax.bitwise_and(pk, 0xFFFF)

    def compute(srcb, msgb, c):
        for off in range(0, CHB, 16):
            wv48[pl.ds(off, 16)] = wb[c, pl.ds(off, 16)]

        def edge(e, carry2):
            wsp = plsc.load_gather(wv48, [jnp.full((16,), e, jnp.int32)])
            for k in range(Q // 32):
                ab = srcb[e, pl.ds(k * 32, 32)]
                a, b = plsc.unpack(ab, format=plsc.PackFormat.INTERLEAVED)
                msgb[e, pl.ds(k * 32, 16)] = a * wsp
                msgb[e, pl.ds(k * 32 + 16, 16)] = b * wsp
            return carry2

        lax.fori_loop(0, CHB, edge, 0, unroll=False)

    def quarter(tsrc, qidx, do_sw):
        def zrow(i, carry):
            for k in range(Q // 16):
                msgb0[i, pl.ds(k * 16, 16)] = jnp.zeros((16,), jnp.float32)
            return carry

        lax.fori_loop(0, CHB, zrow, 0, unroll=False)

        def zcp(z, carry):
            pltpu.sync_copy(msgb0,
                            acc_sh.at[pl.ds(ss * _RPS + z * CHB, CHB)])
            return carry

        lax.fori_loop(0, _RPS // CHB, zcp, 0, unroll=False)
        pltpu.sync_copy(msgb0.at[pl.ds(0, _RPS - (_RPS // CHB) * CHB)],
                        acc_sh.at[pl.ds(ss * _RPS + (_RPS // CHB) * CHB,
                                        _RPS - (_RPS // CHB) * CHB)])
        if do_sw:
            for kz in range(40):
                zb1[pl.ds(kz * 16, 16)] = jnp.zeros((16,), jnp.float32)
            pltpu.sync_copy(zb1, acc_w.at[pl.ds(ss * 624, 640)])
        plsc.subcore_barrier()

        def do_chunk_pair(c0, c1):
            unpack(c0, idxr0, idxc0)
            unpack(c1, idxr1, idxc1)
            h0s = pltpu.async_copy(tsrc.at[idxr0], srcb0, sem0)
            h1s = pltpu.async_copy(tsrc.at[idxr1], srcb1, sem2)
            h0s.wait()
            compute(srcb0, msgb0, c0)
            hsc0 = pltpu.async_copy(msgb0, acc_sh.at[idxc0], add=True,
                                    sem=sem1)
            if do_sw:
                pltpu.sync_copy(wv48.at[pl.ds(0, CHB)],
                                acc_w.at[idxc0], add=True)
            h1s.wait()
            compute(srcb1, msgb1, c1)
            hsc1 = pltpu.async_copy(msgb1, acc_sh.at[idxc1], add=True,
                                    sem=sem3)
            if do_sw:
                pltpu.sync_copy(wv48.at[pl.ds(0, CHB)],
                                acc_w.at[idxc1], add=True)
            hsc0.wait()
            hsc1.wait()

        for half, nch in ((0, 63), (1, 62)):
            base = ss * _STEPS_B + half * 63
            pltpu.sync_copy(pk_hbm.at[pl.ds(base, nch)],
                            pkb.at[pl.ds(0, nch)])
            pltpu.sync_copy(wc_hbm.at[pl.ds(base, nch)],
                            wb.at[pl.ds(0, nch)])

            def pair(i, carry):
                do_chunk_pair(2 * i, 2 * i + 1)
                return carry

            lax.fori_loop(0, nch // 2, pair, 0, unroll=False)
            if nch % 2:
                ce = nch - 1
                unpack(ce, idxr0, idxc0)
                he_s = pltpu.async_copy(tsrc.at[idxr0], srcb0, sem0)
                he_s.wait()
                compute(srcb0, msgb0, ce)
                pltpu.sync_copy(msgb0, acc_sh.at[idxc0], add=True)
                if do_sw:
                    pltpu.sync_copy(wv48.at[pl.ds(0, CHB)],
                                    acc_w.at[idxc0], add=True)
        plsc.subcore_barrier()
        pltpu.sync_copy(
            acc_sh.at[pl.ds(ss * _RPS, _RPS)],
            agg_hbm.at[qidx, pl.ds(ss * _RPS, _RPS)])
        if do_sw:
            pltpu.sync_copy(acc_w.at[pl.ds(ss * 624, 640)],
                            sw_hbm.at[pl.ds(ss * 624, 640)])
        plsc.subcore_barrier()

    @pl.when(cc == 0)
    def _():
        quarter(tm_s0, 0, True)
        quarter(tm_s1, 1, False)

    @pl.when(cc == 1)
    def _():
        quarter(tm_s2, 2, False)
        quarter(tm_s3, 3, False)


def kernel(x, edge_index, W_in, b_in, Wm, bm, Wa1, ba1, Wa2, ba2,
           Wu1, bu1, Wu2, bu2, ln_g, ln_b, out_g, out_b):
    row = edge_index[0]
    col = edge_index[1]
    row_a = row.reshape(E // CHA, CHA)
    col_a = col.reshape(E // CHA, CHA)
    pk_b = (jnp.left_shift(row, 16) | col).reshape(E // CHB, CHB)
    h = _tc_in(x, W_in, b_in)
    pidx = jnp.arange(H).reshape(H // 32, 16, 2).transpose(0, 2, 1).reshape(H)
    for l in range(L):
        wcat = jnp.concatenate(
            [Wm[l][:H], Wm[l][H:][:, pidx], Wa1[l][:H], Wa1[l][H:]], axis=1)
        bcat = jnp.concatenate(
            [jnp.zeros((H,), jnp.float32), bm[l][pidx],
             jnp.zeros((H,), jnp.float32), ba1[l]])
        tm_s, tm_d, ta_s, ta_d = _tc_proj(h, wcat, bcat)
        wa2_re = Wa2[l].reshape(H // 32, 16, 2).transpose(0, 2, 1).reshape(H)
        w = _sc_attn(row_a, col_a, ta_s, ta_d, wa2_re,
                     jnp.full((16,), ba2[l], jnp.float32))
        agg, sw = _sc_agg(pk_b, w.reshape(E // CHB, CHB),
                          tm_s[0], tm_s[1], tm_s[2], tm_s[3])
        h = _tc_update(agg, tm_d, sw, h, Wu1[l][pidx, :], bu1[l],
                       Wu2[l], bu2[l], ln_g[l], ln_b[l])
    return _tc_ln(h, out_g, out_b)
